# Initial kernel scaffold; baseline (speedup 1.0000x reference)
import jax
import jax.numpy as jnp
from jax.experimental import pallas as pl


def kernel(window, params):
    raise NotImplementedError("write your pallas kernel here")


# speedup vs baseline: 26.1832x; 26.1832x over previous
import functools

import jax
import jax.numpy as jnp
from jax.experimental import pallas as pl

N = 2048
IN_C = 128
HID = 256
OUT_C = 128
HEADS = 4
TOP_K = 16
NHEAD = 4
DFF = 512

RB = 256
RBI = 512

F32 = jnp.float32


def _dot(a, b):
    return jnp.dot(a, b, preferred_element_type=F32)


def _emb_kernel(x_ref, w_ref, out_ref):
    out_ref[0] = jnp.tanh(_dot(x_ref[...], w_ref[0]))


def _topk_kernel(embr_ref, embf_ref, idx_ref):
    er = embr_ref[0]
    ef = embf_ref[0]
    sim = jax.lax.dot_general(er, ef, (((1,), (1,)), ((), ())),
                              preferred_element_type=F32)
    cols = jax.lax.broadcasted_iota(jnp.int32, (RB, N), 1)
    lane = jax.lax.broadcasted_iota(jnp.int32, (RB, 128), 1)
    s = sim
    acc = jnp.zeros((RB, 128), jnp.int32)
    for k in range(TOP_K):
        m = jnp.max(s, axis=1, keepdims=True)
        ik = jnp.min(jnp.where(s == m, cols, N), axis=1, keepdims=True)
        acc = jnp.where(lane == k, ik, acc)
        s = jnp.where(cols == ik, -jnp.inf, s)
    idx_ref[0] = acc


def _gat_in_kernel(x_ref, w_ref, a_ref, h_ref, ab_ref):
    h = _dot(x_ref[...], w_ref[...])
    h_ref[...] = h
    ab_ref[...] = _dot(h, a_ref[...])


def _gat_agg_kernel(idx_ref, h_ref, abT_ref, ab_ref, b_ref, o_ref, *,
                    heads, ch, relu):
    i = pl.program_id(0)
    cols = jax.lax.broadcasted_iota(jnp.int32, (RB, N), 1)
    rowid = i * RB + jax.lax.broadcasted_iota(jnp.int32, (RB, 1), 0)
    B = (cols == rowid).astype(F32)
    idx = idx_ref[...]
    for j in range(TOP_K):
        B = B + (cols == idx[:, j:j + 1]).astype(F32)
    hf = h_ref[...]
    outs = []
    for hd in range(heads):
        asrc = abT_ref[hd:hd + 1, :]
        adst = ab_ref[:, 64 + hd:65 + hd]
        d = asrc + adst
        d = jnp.where(d >= 0, d, 0.2 * d)
        dm = jnp.where(B > 0, d, -jnp.inf)
        m = jnp.max(dm, axis=1, keepdims=True)
        e = jnp.exp(dm - m) * B
        ssum = jnp.sum(e, axis=1, keepdims=True)
        attn = e / (ssum + 1e-16)
        outs.append(_dot(attn, hf[:, hd * ch:(hd + 1) * ch]))
    o = jnp.concatenate(outs, axis=1) if heads > 1 else outs[0]
    o = o + b_ref[...]
    if relu:
        o = jnp.maximum(o, 0.0)
    o_ref[...] = o


def _gat_layer(x, idxp_l, p, heads, ch, relu):
    cin = x.shape[1]
    cout = heads * ch
    att_src = p['att_src']
    att_dst = p['att_dst']
    amat = jnp.zeros((cout, 128), F32)
    for hd in range(heads):
        amat = amat.at[hd * ch:(hd + 1) * ch, hd].set(att_src[hd])
        amat = amat.at[hd * ch:(hd + 1) * ch, 64 + hd].set(att_dst[hd])
    h, ab = pl.pallas_call(
        _gat_in_kernel,
        grid=(N // RBI,),
        in_specs=[
            pl.BlockSpec((RBI, cin), lambda r: (r, 0)),
            pl.BlockSpec((cin, cout), lambda r: (0, 0)),
            pl.BlockSpec((cout, 128), lambda r: (0, 0)),
        ],
        out_specs=[
            pl.BlockSpec((RBI, cout), lambda r: (r, 0)),
            pl.BlockSpec((RBI, 128), lambda r: (r, 0)),
        ],
        out_shape=[
            jax.ShapeDtypeStruct((N, cout), F32),
            jax.ShapeDtypeStruct((N, 128), F32),
        ],
    )(x, p['W'], amat)
    abT = ab.T
    bias2 = p['bias'][None, :]
    out = pl.pallas_call(
        functools.partial(_gat_agg_kernel, heads=heads, ch=ch, relu=relu),
        grid=(N // RB,),
        in_specs=[
            pl.BlockSpec((RB, 128), lambda r: (r, 0)),
            pl.BlockSpec((N, cout), lambda r: (0, 0)),
            pl.BlockSpec((128, N), lambda r: (0, 0)),
            pl.BlockSpec((RB, 128), lambda r: (r, 0)),
            pl.BlockSpec((1, cout), lambda r: (0, 0)),
        ],
        out_specs=pl.BlockSpec((RB, cout), lambda r: (r, 0)),
        out_shape=jax.ShapeDtypeStruct((N, cout), F32),
    )(idxp_l, h, abT, ab, bias2)
    return out


def _mm_bias_kernel(x_ref, w_ref, b_ref, o_ref):
    o_ref[...] = _dot(x_ref[...], w_ref[...]) + b_ref[...]


def _attn_kernel(q_ref, k_ref, v_ref, o_ref):
    q = q_ref[...]
    k = k_ref[...]
    v = v_ref[...]
    dh = HID // NHEAD
    outs = []
    for hd in range(NHEAD):
        sl = slice(hd * dh, (hd + 1) * dh)
        s = jax.lax.dot_general(q[:, sl], k[:, sl], (((1,), (1,)), ((), ())),
                                preferred_element_type=F32) * 0.125
        m = jnp.max(s, axis=1, keepdims=True)
        e = jnp.exp(s - m)
        p = e / jnp.sum(e, axis=1, keepdims=True)
        outs.append(_dot(p, v[:, sl]))
    o_ref[...] = jnp.concatenate(outs, axis=1)


def _ln(x, g, b):
    m = jnp.mean(x, axis=-1, keepdims=True)
    v = jnp.mean((x - m) * (x - m), axis=-1, keepdims=True)
    return (x - m) / jnp.sqrt(v + 1e-5) * g + b


def _post_kernel(x_ref, o_ref, ow_ref, ob_ref, g1_ref, b1_ref, w1_ref,
                 bb1_ref, w2_ref, bb2_ref, g2_ref, b2_ref, win_ref, skw_ref,
                 skb_ref, out_ref, *, skip):
    x = x_ref[...]
    a = _dot(o_ref[...], ow_ref[...]) + ob_ref[...]
    x1 = _ln(x + a, g1_ref[...], b1_ref[...])
    f = jnp.maximum(_dot(x1, w1_ref[...]) + bb1_ref[...], 0.0)
    f = _dot(f, w2_ref[...]) + bb2_ref[...]
    x2 = _ln(x1 + f, g2_ref[...], b2_ref[...])
    if skip:
        x2 = x2 + _dot(win_ref[...], skw_ref[...]) + skb_ref[...]
    out_ref[...] = x2


def _trans_layer(x, p, window, skw, skb, skip):
    qkv = pl.pallas_call(
        _mm_bias_kernel,
        grid=(N // RBI,),
        in_specs=[
            pl.BlockSpec((RBI, HID), lambda r: (r, 0)),
            pl.BlockSpec((HID, 3 * HID), lambda r: (0, 0)),
            pl.BlockSpec((1, 3 * HID), lambda r: (0, 0)),
        ],
        out_specs=pl.BlockSpec((RBI, 3 * HID), lambda r: (r, 0)),
        out_shape=jax.ShapeDtypeStruct((N, 3 * HID), F32),
    )(x, p['in_w'].T, p['in_b'][None, :])
    o = pl.pallas_call(
        _attn_kernel,
        grid=(N // RBI,),
        in_specs=[
            pl.BlockSpec((RBI, HID), lambda r: (r, 0)),
            pl.BlockSpec((N, HID), lambda r: (0, 1)),
            pl.BlockSpec((N, HID), lambda r: (0, 2)),
        ],
        out_specs=pl.BlockSpec((RBI, HID), lambda r: (r, 0)),
        out_shape=jax.ShapeDtypeStruct((N, HID), F32),
    )(qkv, qkv, qkv)
    out = pl.pallas_call(
        functools.partial(_post_kernel, skip=skip),
        grid=(N // RBI,),
        in_specs=[
            pl.BlockSpec((RBI, HID), lambda r: (r, 0)),
            pl.BlockSpec((RBI, HID), lambda r: (r, 0)),
            pl.BlockSpec((HID, HID), lambda r: (0, 0)),
            pl.BlockSpec((1, HID), lambda r: (0, 0)),
            pl.BlockSpec((1, HID), lambda r: (0, 0)),
            pl.BlockSpec((1, HID), lambda r: (0, 0)),
            pl.BlockSpec((HID, DFF), lambda r: (0, 0)),
            pl.BlockSpec((1, DFF), lambda r: (0, 0)),
            pl.BlockSpec((DFF, HID), lambda r: (0, 0)),
            pl.BlockSpec((1, HID), lambda r: (0, 0)),
            pl.BlockSpec((1, HID), lambda r: (0, 0)),
            pl.BlockSpec((1, HID), lambda r: (0, 0)),
            pl.BlockSpec((RBI, IN_C), lambda r: (r, 0)),
            pl.BlockSpec((IN_C, HID), lambda r: (0, 0)),
            pl.BlockSpec((1, HID), lambda r: (0, 0)),
        ],
        out_specs=pl.BlockSpec((RBI, HID), lambda r: (r, 0)),
        out_shape=jax.ShapeDtypeStruct((N, HID), F32),
    )(x, o, p['out_w'].T, p['out_b'][None, :], p['ln1_g'][None, :],
      p['ln1_b'][None, :], p['l1_w'].T, p['l1_b'][None, :], p['l2_w'].T,
      p['l2_b'][None, :], p['ln2_g'][None, :], p['ln2_b'][None, :],
      window, skw, skb)
    return out


def kernel(window, params):
    x = window
    gl_w = jnp.stack(params['gl_W'])
    nl = gl_w.shape[0]
    emb = pl.pallas_call(
        _emb_kernel,
        grid=(nl,),
        in_specs=[
            pl.BlockSpec((N, IN_C), lambda l: (0, 0)),
            pl.BlockSpec((1, IN_C, HID), lambda l: (l, 0, 0)),
        ],
        out_specs=pl.BlockSpec((1, N, HID), lambda l: (l, 0, 0)),
        out_shape=jax.ShapeDtypeStruct((nl, N, HID), F32),
    )(x, gl_w)
    idxp = pl.pallas_call(
        _topk_kernel,
        grid=(nl, N // RB),
        in_specs=[
            pl.BlockSpec((1, RB, HID), lambda l, r: (l, r, 0)),
            pl.BlockSpec((1, N, HID), lambda l, r: (l, 0, 0)),
        ],
        out_specs=pl.BlockSpec((1, RB, 128), lambda l, r: (l, r, 0)),
        out_shape=jax.ShapeDtypeStruct((nl, N, 128), jnp.int32),
    )(emb, emb)

    h = x
    for i, p in enumerate(params['enc']):
        h = _gat_layer(h, idxp[i], p, HEADS, HID // HEADS, relu=True)

    skw = params['skip_w'].T
    skb = params['skip_b'][None, :]
    ht = h
    for li, p in enumerate(params['trans']):
        ht = _trans_layer(ht, p, window, skw, skb,
                          skip=(li == len(params['trans']) - 1))

    d = ht
    dec = params['dec']
    for i in range(len(dec) - 1):
        d = _gat_layer(d, idxp[3 + i], dec[i], HEADS, HID // HEADS, relu=True)
    d = _gat_layer(d, idxp[3 + len(dec) - 1], dec[-1], 1, OUT_C, relu=False)
    return d.T


# speedup vs baseline: 28.6223x; 1.0932x over previous
import functools

import jax
import jax.numpy as jnp
from jax.experimental import pallas as pl

N = 2048
IN_C = 128
HID = 256
OUT_C = 128
HEADS = 4
TOP_K = 16
NHEAD = 4
DFF = 512

RB = 256
RBI = 512

F32 = jnp.float32


def _dot(a, b):
    return jnp.dot(a, b, preferred_element_type=F32)


def _emb_kernel(x_ref, w_ref, out_ref):
    out_ref[0] = jnp.tanh(_dot(x_ref[...], w_ref[0]))


def _topk_kernel(embr_ref, embf_ref, idx_ref):
    er = embr_ref[0]
    ef = embf_ref[0]
    sim = jax.lax.dot_general(er, ef, (((1,), (1,)), ((), ())),
                              preferred_element_type=F32)
    cols = jax.lax.broadcasted_iota(jnp.int32, (RB, N), 1)
    lane = jax.lax.broadcasted_iota(jnp.int32, (RB, 128), 1)
    s = sim
    acc = jnp.zeros((RB, 128), jnp.int32)
    for k in range(TOP_K):
        m = jnp.max(s, axis=1, keepdims=True)
        ik = jnp.min(jnp.where(s == m, cols, N), axis=1, keepdims=True)
        acc = jnp.where(lane == k, ik, acc)
        s = jnp.where(cols == ik, -jnp.inf, s)
    idx_ref[0] = acc


def _gat_in_kernel(x_ref, w_ref, a_ref, h_ref, ab_ref):
    h = _dot(x_ref[...], w_ref[...])
    h_ref[...] = h
    ab_ref[...] = _dot(h, a_ref[...])


def _gat_agg_kernel(idx_ref, h_ref, abT_ref, ab_ref, b_ref, o_ref, *,
                    heads, ch, relu):
    i = pl.program_id(0)
    cols = jax.lax.broadcasted_iota(jnp.int32, (RB, N), 1)
    rowid = i * RB + jax.lax.broadcasted_iota(jnp.int32, (RB, 1), 0)
    B = (cols == rowid).astype(F32)
    idx = idx_ref[...]
    for j in range(TOP_K):
        B = B + (cols == idx[:, j:j + 1]).astype(F32)
    hf = h_ref[...]
    outs = []
    for hd in range(heads):
        asrc = abT_ref[hd:hd + 1, :]
        adst = ab_ref[:, 64 + hd:65 + hd]
        d = asrc + adst
        d = jnp.where(d >= 0, d, 0.2 * d)
        e = jnp.exp(d) * B
        ssum = jnp.sum(e, axis=1, keepdims=True)
        outs.append(_dot(e, hf[:, hd * ch:(hd + 1) * ch]) / (ssum + 1e-16))
    o = jnp.concatenate(outs, axis=1) if heads > 1 else outs[0]
    o = o + b_ref[...]
    if relu:
        o = jnp.maximum(o, 0.0)
    o_ref[...] = o


def _gat_layer(x, idxp_l, p, heads, ch, relu):
    cin = x.shape[1]
    cout = heads * ch
    att_src = p['att_src']
    att_dst = p['att_dst']
    amat = jnp.zeros((cout, 128), F32)
    for hd in range(heads):
        amat = amat.at[hd * ch:(hd + 1) * ch, hd].set(att_src[hd])
        amat = amat.at[hd * ch:(hd + 1) * ch, 64 + hd].set(att_dst[hd])
    h, ab = pl.pallas_call(
        _gat_in_kernel,
        grid=(N // RBI,),
        in_specs=[
            pl.BlockSpec((RBI, cin), lambda r: (r, 0)),
            pl.BlockSpec((cin, cout), lambda r: (0, 0)),
            pl.BlockSpec((cout, 128), lambda r: (0, 0)),
        ],
        out_specs=[
            pl.BlockSpec((RBI, cout), lambda r: (r, 0)),
            pl.BlockSpec((RBI, 128), lambda r: (r, 0)),
        ],
        out_shape=[
            jax.ShapeDtypeStruct((N, cout), F32),
            jax.ShapeDtypeStruct((N, 128), F32),
        ],
    )(x, p['W'], amat)
    abT = ab.T
    bias2 = p['bias'][None, :]
    out = pl.pallas_call(
        functools.partial(_gat_agg_kernel, heads=heads, ch=ch, relu=relu),
        grid=(N // RB,),
        in_specs=[
            pl.BlockSpec((RB, 128), lambda r: (r, 0)),
            pl.BlockSpec((N, cout), lambda r: (0, 0)),
            pl.BlockSpec((128, N), lambda r: (0, 0)),
            pl.BlockSpec((RB, 128), lambda r: (r, 0)),
            pl.BlockSpec((1, cout), lambda r: (0, 0)),
        ],
        out_specs=pl.BlockSpec((RB, cout), lambda r: (r, 0)),
        out_shape=jax.ShapeDtypeStruct((N, cout), F32),
    )(idxp_l, h, abT, ab, bias2)
    return out


def _mm_bias_kernel(x_ref, w_ref, b_ref, o_ref):
    o_ref[...] = _dot(x_ref[...], w_ref[...]) + b_ref[...]


def _attn_kernel(q_ref, k_ref, v_ref, o_ref):
    q = q_ref[...]
    k = k_ref[...]
    v = v_ref[...]
    dh = HID // NHEAD
    outs = []
    for hd in range(NHEAD):
        sl = slice(hd * dh, (hd + 1) * dh)
        s = jax.lax.dot_general(q[:, sl], k[:, sl], (((1,), (1,)), ((), ())),
                                preferred_element_type=F32) * 0.125
        e = jnp.exp(s)
        ssum = jnp.sum(e, axis=1, keepdims=True)
        outs.append(_dot(e, v[:, sl]) / ssum)
    o_ref[...] = jnp.concatenate(outs, axis=1)


def _ln(x, g, b):
    m = jnp.mean(x, axis=-1, keepdims=True)
    v = jnp.mean((x - m) * (x - m), axis=-1, keepdims=True)
    return (x - m) / jnp.sqrt(v + 1e-5) * g + b


def _post_kernel(x_ref, o_ref, ow_ref, ob_ref, g1_ref, b1_ref, w1_ref,
                 bb1_ref, w2_ref, bb2_ref, g2_ref, b2_ref, win_ref, skw_ref,
                 skb_ref, out_ref, *, skip):
    x = x_ref[...]
    a = _dot(o_ref[...], ow_ref[...]) + ob_ref[...]
    x1 = _ln(x + a, g1_ref[...], b1_ref[...])
    f = jnp.maximum(_dot(x1, w1_ref[...]) + bb1_ref[...], 0.0)
    f = _dot(f, w2_ref[...]) + bb2_ref[...]
    x2 = _ln(x1 + f, g2_ref[...], b2_ref[...])
    if skip:
        x2 = x2 + _dot(win_ref[...], skw_ref[...]) + skb_ref[...]
    out_ref[...] = x2


def _trans_layer(x, p, window, skw, skb, skip):
    qkv = pl.pallas_call(
        _mm_bias_kernel,
        grid=(N // RBI,),
        in_specs=[
            pl.BlockSpec((RBI, HID), lambda r: (r, 0)),
            pl.BlockSpec((HID, 3 * HID), lambda r: (0, 0)),
            pl.BlockSpec((1, 3 * HID), lambda r: (0, 0)),
        ],
        out_specs=pl.BlockSpec((RBI, 3 * HID), lambda r: (r, 0)),
        out_shape=jax.ShapeDtypeStruct((N, 3 * HID), F32),
    )(x, p['in_w'].T, p['in_b'][None, :])
    o = pl.pallas_call(
        _attn_kernel,
        grid=(N // RBI,),
        in_specs=[
            pl.BlockSpec((RBI, HID), lambda r: (r, 0)),
            pl.BlockSpec((N, HID), lambda r: (0, 1)),
            pl.BlockSpec((N, HID), lambda r: (0, 2)),
        ],
        out_specs=pl.BlockSpec((RBI, HID), lambda r: (r, 0)),
        out_shape=jax.ShapeDtypeStruct((N, HID), F32),
    )(qkv, qkv, qkv)
    out = pl.pallas_call(
        functools.partial(_post_kernel, skip=skip),
        grid=(N // RBI,),
        in_specs=[
            pl.BlockSpec((RBI, HID), lambda r: (r, 0)),
            pl.BlockSpec((RBI, HID), lambda r: (r, 0)),
            pl.BlockSpec((HID, HID), lambda r: (0, 0)),
            pl.BlockSpec((1, HID), lambda r: (0, 0)),
            pl.BlockSpec((1, HID), lambda r: (0, 0)),
            pl.BlockSpec((1, HID), lambda r: (0, 0)),
            pl.BlockSpec((HID, DFF), lambda r: (0, 0)),
            pl.BlockSpec((1, DFF), lambda r: (0, 0)),
            pl.BlockSpec((DFF, HID), lambda r: (0, 0)),
            pl.BlockSpec((1, HID), lambda r: (0, 0)),
            pl.BlockSpec((1, HID), lambda r: (0, 0)),
            pl.BlockSpec((1, HID), lambda r: (0, 0)),
            pl.BlockSpec((RBI, IN_C), lambda r: (r, 0)),
            pl.BlockSpec((IN_C, HID), lambda r: (0, 0)),
            pl.BlockSpec((1, HID), lambda r: (0, 0)),
        ],
        out_specs=pl.BlockSpec((RBI, HID), lambda r: (r, 0)),
        out_shape=jax.ShapeDtypeStruct((N, HID), F32),
    )(x, o, p['out_w'].T, p['out_b'][None, :], p['ln1_g'][None, :],
      p['ln1_b'][None, :], p['l1_w'].T, p['l1_b'][None, :], p['l2_w'].T,
      p['l2_b'][None, :], p['ln2_g'][None, :], p['ln2_b'][None, :],
      window, skw, skb)
    return out


def kernel(window, params):
    x = window
    gl_w = jnp.stack(params['gl_W'])
    nl = gl_w.shape[0]
    emb = pl.pallas_call(
        _emb_kernel,
        grid=(nl,),
        in_specs=[
            pl.BlockSpec((N, IN_C), lambda l: (0, 0)),
            pl.BlockSpec((1, IN_C, HID), lambda l: (l, 0, 0)),
        ],
        out_specs=pl.BlockSpec((1, N, HID), lambda l: (l, 0, 0)),
        out_shape=jax.ShapeDtypeStruct((nl, N, HID), F32),
    )(x, gl_w)
    idxp = pl.pallas_call(
        _topk_kernel,
        grid=(nl, N // RB),
        in_specs=[
            pl.BlockSpec((1, RB, HID), lambda l, r: (l, r, 0)),
            pl.BlockSpec((1, N, HID), lambda l, r: (l, 0, 0)),
        ],
        out_specs=pl.BlockSpec((1, RB, 128), lambda l, r: (l, r, 0)),
        out_shape=jax.ShapeDtypeStruct((nl, N, 128), jnp.int32),
    )(emb, emb)

    h = x
    for i, p in enumerate(params['enc']):
        h = _gat_layer(h, idxp[i], p, HEADS, HID // HEADS, relu=True)

    skw = params['skip_w'].T
    skb = params['skip_b'][None, :]
    ht = h
    for li, p in enumerate(params['trans']):
        ht = _trans_layer(ht, p, window, skw, skb,
                          skip=(li == len(params['trans']) - 1))

    d = ht
    dec = params['dec']
    for i in range(len(dec) - 1):
        d = _gat_layer(d, idxp[3 + i], dec[i], HEADS, HID // HEADS, relu=True)
    d = _gat_layer(d, idxp[3 + len(dec) - 1], dec[-1], 1, OUT_C, relu=False)
    return d.T


# speedup vs baseline: 48.2283x; 1.6850x over previous
import functools

import jax
import jax.numpy as jnp
from jax.experimental import pallas as pl

N = 2048
IN_C = 128
HID = 256
OUT_C = 128
HEADS = 4
TOP_K = 16
NHEAD = 4
DFF = 512

RB = 256
RBI = 512

F32 = jnp.float32


def _dot(a, b):
    return jnp.dot(a, b, preferred_element_type=F32)


def _emb_kernel(x_ref, w_ref, out_ref):
    out_ref[0] = jnp.tanh(_dot(x_ref[...], w_ref[0]))


def _thresh_kernel(embr_ref, embf_ref, thr_ref):
    er = embr_ref[0]
    ef = embf_ref[0]
    sim = jax.lax.dot_general(er, ef, (((1,), (1,)), ((), ())),
                              preferred_element_type=F32)
    s = sim
    m = None
    m_prev = None
    for k in range(TOP_K + 1):
        m_prev = m
        m = jnp.max(s, axis=1, keepdims=True)
        if k < TOP_K:
            s = jnp.where(s < m, s, -jnp.inf)
    t = (m_prev + m) * 0.5
    thr_ref[0] = jnp.broadcast_to(t, (RB, 128))


def _gat_in_kernel(x_ref, w_ref, a_ref, h_ref, ab_ref):
    h = _dot(x_ref[...], w_ref[...])
    h_ref[...] = h
    ab_ref[...] = _dot(h, a_ref[...])


def _gat_agg_kernel(embr_ref, embf_ref, thr_ref, h_ref, abT_ref, ab_ref,
                    b_ref, o_ref, *, heads, ch, relu):
    i = pl.program_id(0)
    cols = jax.lax.broadcasted_iota(jnp.int32, (RB, N), 1)
    rowid = i * RB + jax.lax.broadcasted_iota(jnp.int32, (RB, 1), 0)
    sim = jax.lax.dot_general(embr_ref[0], embf_ref[0],
                              (((1,), (1,)), ((), ())),
                              preferred_element_type=F32)
    t = thr_ref[:, 0:1]
    B = (sim >= t).astype(F32) + (cols == rowid).astype(F32)
    hf = h_ref[...]
    outs = []
    for hd in range(heads):
        asrc = abT_ref[hd:hd + 1, :]
        adst = ab_ref[:, 64 + hd:65 + hd]
        d = asrc + adst
        d = jnp.where(d >= 0, d, 0.2 * d)
        e = jnp.exp(d) * B
        ssum = jnp.sum(e, axis=1, keepdims=True)
        outs.append(_dot(e, hf[:, hd * ch:(hd + 1) * ch]) / (ssum + 1e-16))
    o = jnp.concatenate(outs, axis=1) if heads > 1 else outs[0]
    o = o + b_ref[...]
    if relu:
        o = jnp.maximum(o, 0.0)
    o_ref[...] = o


def _gat_layer(x, emb, thr, li, p, heads, ch, relu):
    cin = x.shape[1]
    cout = heads * ch
    att_src = p['att_src']
    att_dst = p['att_dst']
    amat = jnp.zeros((cout, 128), F32)
    for hd in range(heads):
        amat = amat.at[hd * ch:(hd + 1) * ch, hd].set(att_src[hd])
        amat = amat.at[hd * ch:(hd + 1) * ch, 64 + hd].set(att_dst[hd])
    h, ab = pl.pallas_call(
        _gat_in_kernel,
        grid=(N // RBI,),
        in_specs=[
            pl.BlockSpec((RBI, cin), lambda r: (r, 0)),
            pl.BlockSpec((cin, cout), lambda r: (0, 0)),
            pl.BlockSpec((cout, 128), lambda r: (0, 0)),
        ],
        out_specs=[
            pl.BlockSpec((RBI, cout), lambda r: (r, 0)),
            pl.BlockSpec((RBI, 128), lambda r: (r, 0)),
        ],
        out_shape=[
            jax.ShapeDtypeStruct((N, cout), F32),
            jax.ShapeDtypeStruct((N, 128), F32),
        ],
    )(x, p['W'], amat)
    abT = ab.T
    bias2 = p['bias'][None, :]
    out = pl.pallas_call(
        functools.partial(_gat_agg_kernel, heads=heads, ch=ch, relu=relu),
        grid=(N // RB,),
        in_specs=[
            pl.BlockSpec((1, RB, HID), lambda r: (li, r, 0)),
            pl.BlockSpec((1, N, HID), lambda r: (li, 0, 0)),
            pl.BlockSpec((RB, 128), lambda r: (r, 0)),
            pl.BlockSpec((N, cout), lambda r: (0, 0)),
            pl.BlockSpec((128, N), lambda r: (0, 0)),
            pl.BlockSpec((RB, 128), lambda r: (r, 0)),
            pl.BlockSpec((1, cout), lambda r: (0, 0)),
        ],
        out_specs=pl.BlockSpec((RB, cout), lambda r: (r, 0)),
        out_shape=jax.ShapeDtypeStruct((N, cout), F32),
    )(emb, emb, thr[li], h, abT, ab, bias2)
    return out


def _mm_bias_kernel(x_ref, w_ref, b_ref, o_ref):
    o_ref[...] = _dot(x_ref[...], w_ref[...]) + b_ref[...]


def _attn_kernel(q_ref, k_ref, v_ref, o_ref):
    q = q_ref[...]
    k = k_ref[...]
    v = v_ref[...]
    dh = HID // NHEAD
    outs = []
    for hd in range(NHEAD):
        sl = slice(hd * dh, (hd + 1) * dh)
        s = jax.lax.dot_general(q[:, sl], k[:, sl], (((1,), (1,)), ((), ())),
                                preferred_element_type=F32) * 0.125
        e = jnp.exp(s)
        ssum = jnp.sum(e, axis=1, keepdims=True)
        outs.append(_dot(e, v[:, sl]) / ssum)
    o_ref[...] = jnp.concatenate(outs, axis=1)


def _ln(x, g, b):
    m = jnp.mean(x, axis=-1, keepdims=True)
    v = jnp.mean((x - m) * (x - m), axis=-1, keepdims=True)
    return (x - m) / jnp.sqrt(v + 1e-5) * g + b


def _post_kernel(x_ref, o_ref, ow_ref, ob_ref, g1_ref, b1_ref, w1_ref,
                 bb1_ref, w2_ref, bb2_ref, g2_ref, b2_ref, win_ref, skw_ref,
                 skb_ref, out_ref, *, skip):
    x = x_ref[...]
    a = _dot(o_ref[...], ow_ref[...]) + ob_ref[...]
    x1 = _ln(x + a, g1_ref[...], b1_ref[...])
    f = jnp.maximum(_dot(x1, w1_ref[...]) + bb1_ref[...], 0.0)
    f = _dot(f, w2_ref[...]) + bb2_ref[...]
    x2 = _ln(x1 + f, g2_ref[...], b2_ref[...])
    if skip:
        x2 = x2 + _dot(win_ref[...], skw_ref[...]) + skb_ref[...]
    out_ref[...] = x2


def _trans_layer(x, p, window, skw, skb, skip):
    qkv = pl.pallas_call(
        _mm_bias_kernel,
        grid=(N // RBI,),
        in_specs=[
            pl.BlockSpec((RBI, HID), lambda r: (r, 0)),
            pl.BlockSpec((HID, 3 * HID), lambda r: (0, 0)),
            pl.BlockSpec((1, 3 * HID), lambda r: (0, 0)),
        ],
        out_specs=pl.BlockSpec((RBI, 3 * HID), lambda r: (r, 0)),
        out_shape=jax.ShapeDtypeStruct((N, 3 * HID), F32),
    )(x, p['in_w'].T, p['in_b'][None, :])
    o = pl.pallas_call(
        _attn_kernel,
        grid=(N // RBI,),
        in_specs=[
            pl.BlockSpec((RBI, HID), lambda r: (r, 0)),
            pl.BlockSpec((N, HID), lambda r: (0, 1)),
            pl.BlockSpec((N, HID), lambda r: (0, 2)),
        ],
        out_specs=pl.BlockSpec((RBI, HID), lambda r: (r, 0)),
        out_shape=jax.ShapeDtypeStruct((N, HID), F32),
    )(qkv, qkv, qkv)
    out = pl.pallas_call(
        functools.partial(_post_kernel, skip=skip),
        grid=(N // RBI,),
        in_specs=[
            pl.BlockSpec((RBI, HID), lambda r: (r, 0)),
            pl.BlockSpec((RBI, HID), lambda r: (r, 0)),
            pl.BlockSpec((HID, HID), lambda r: (0, 0)),
            pl.BlockSpec((1, HID), lambda r: (0, 0)),
            pl.BlockSpec((1, HID), lambda r: (0, 0)),
            pl.BlockSpec((1, HID), lambda r: (0, 0)),
            pl.BlockSpec((HID, DFF), lambda r: (0, 0)),
            pl.BlockSpec((1, DFF), lambda r: (0, 0)),
            pl.BlockSpec((DFF, HID), lambda r: (0, 0)),
            pl.BlockSpec((1, HID), lambda r: (0, 0)),
            pl.BlockSpec((1, HID), lambda r: (0, 0)),
            pl.BlockSpec((1, HID), lambda r: (0, 0)),
            pl.BlockSpec((RBI, IN_C), lambda r: (r, 0)),
            pl.BlockSpec((IN_C, HID), lambda r: (0, 0)),
            pl.BlockSpec((1, HID), lambda r: (0, 0)),
        ],
        out_specs=pl.BlockSpec((RBI, HID), lambda r: (r, 0)),
        out_shape=jax.ShapeDtypeStruct((N, HID), F32),
    )(x, o, p['out_w'].T, p['out_b'][None, :], p['ln1_g'][None, :],
      p['ln1_b'][None, :], p['l1_w'].T, p['l1_b'][None, :], p['l2_w'].T,
      p['l2_b'][None, :], p['ln2_g'][None, :], p['ln2_b'][None, :],
      window, skw, skb)
    return out


def kernel(window, params):
    x = window
    gl_w = jnp.stack(params['gl_W'])
    nl = gl_w.shape[0]
    emb = pl.pallas_call(
        _emb_kernel,
        grid=(nl,),
        in_specs=[
            pl.BlockSpec((N, IN_C), lambda l: (0, 0)),
            pl.BlockSpec((1, IN_C, HID), lambda l: (l, 0, 0)),
        ],
        out_specs=pl.BlockSpec((1, N, HID), lambda l: (l, 0, 0)),
        out_shape=jax.ShapeDtypeStruct((nl, N, HID), F32),
    )(x, gl_w)
    thr = pl.pallas_call(
        _thresh_kernel,
        grid=(nl, N // RB),
        in_specs=[
            pl.BlockSpec((1, RB, HID), lambda l, r: (l, r, 0)),
            pl.BlockSpec((1, N, HID), lambda l, r: (l, 0, 0)),
        ],
        out_specs=pl.BlockSpec((1, RB, 128), lambda l, r: (l, r, 0)),
        out_shape=jax.ShapeDtypeStruct((nl, N, 128), F32),
    )(emb, emb)

    h = x
    for i, p in enumerate(params['enc']):
        h = _gat_layer(h, emb, thr, i, p, HEADS, HID // HEADS, relu=True)

    skw = params['skip_w'].T
    skb = params['skip_b'][None, :]
    ht = h
    for li, p in enumerate(params['trans']):
        ht = _trans_layer(ht, p, window, skw, skb,
                          skip=(li == len(params['trans']) - 1))

    d = ht
    dec = params['dec']
    for i in range(len(dec) - 1):
        d = _gat_layer(d, emb, thr, 3 + i, dec[i], HEADS, HID // HEADS,
                       relu=True)
    d = _gat_layer(d, emb, thr, 3 + len(dec) - 1, dec[-1], 1, OUT_C,
                   relu=False)
    return d.T


# speedup vs baseline: 57.3398x; 1.1889x over previous
import functools

import jax
import jax.numpy as jnp
from jax.experimental import pallas as pl

N = 2048
IN_C = 128
HID = 256
OUT_C = 128
HEADS = 4
TOP_K = 16
NHEAD = 4
DFF = 512

RB = 256
RBI = 512

F32 = jnp.float32


def _dot(a, b):
    return jnp.dot(a, b, preferred_element_type=F32)


def _emb_kernel(x_ref, w_ref, out_ref):
    out_ref[0] = jnp.tanh(_dot(x_ref[...], w_ref[0]))


def _thresh_kernel(embr_ref, embf_ref, thr_ref):
    er = embr_ref[0]
    ef = embf_ref[0]
    sim = jax.lax.dot_general(er, ef, (((1,), (1,)), ((), ())),
                              preferred_element_type=F32)
    a = sim[:, 0:128]
    b = jnp.full((RB, 128), -jnp.inf, F32)
    c = b
    for ci in range(1, N // 128):
        x = sim[:, ci * 128:(ci + 1) * 128]
        lo = jnp.minimum(a, x)
        a = jnp.maximum(a, x)
        lo2 = jnp.minimum(b, lo)
        b = jnp.maximum(b, lo)
        c = jnp.maximum(c, lo2)
    w = jnp.concatenate([a, b, c], axis=1)
    m = None
    m_prev = None
    for k in range(TOP_K + 1):
        m_prev = m
        m = jnp.max(w, axis=1, keepdims=True)
        if k < TOP_K:
            w = jnp.where(w < m, w, -jnp.inf)
    t = (m_prev + m) * 0.5
    thr_ref[0] = jnp.broadcast_to(t, (RB, 128))


def _gat_in_kernel(x_ref, w_ref, a_ref, h_ref, ab_ref):
    h = _dot(x_ref[...], w_ref[...])
    h_ref[...] = h
    ab_ref[...] = _dot(h, a_ref[...])


def _gat_agg_kernel(embr_ref, embf_ref, thr_ref, h_ref, abT_ref, ab_ref,
                    b_ref, o_ref, *, heads, ch, relu):
    i = pl.program_id(0)
    cols = jax.lax.broadcasted_iota(jnp.int32, (RB, N), 1)
    rowid = i * RB + jax.lax.broadcasted_iota(jnp.int32, (RB, 1), 0)
    sim = jax.lax.dot_general(embr_ref[0], embf_ref[0],
                              (((1,), (1,)), ((), ())),
                              preferred_element_type=F32)
    t = thr_ref[:, 0:1]
    B = (sim >= t).astype(F32) + (cols == rowid).astype(F32)
    hf = h_ref[...]
    outs = []
    for hd in range(heads):
        asrc = abT_ref[hd:hd + 1, :]
        adst = ab_ref[:, 64 + hd:65 + hd]
        d = asrc + adst
        d = jnp.where(d >= 0, d, 0.2 * d)
        e = jnp.exp(d) * B
        ssum = jnp.sum(e, axis=1, keepdims=True)
        outs.append(_dot(e, hf[:, hd * ch:(hd + 1) * ch]) / (ssum + 1e-16))
    o = jnp.concatenate(outs, axis=1) if heads > 1 else outs[0]
    o = o + b_ref[...]
    if relu:
        o = jnp.maximum(o, 0.0)
    o_ref[...] = o


def _gat_layer(x, emb, thr, li, p, heads, ch, relu):
    cin = x.shape[1]
    cout = heads * ch
    att_src = p['att_src']
    att_dst = p['att_dst']
    amat = jnp.zeros((cout, 128), F32)
    for hd in range(heads):
        amat = amat.at[hd * ch:(hd + 1) * ch, hd].set(att_src[hd])
        amat = amat.at[hd * ch:(hd + 1) * ch, 64 + hd].set(att_dst[hd])
    h, ab = pl.pallas_call(
        _gat_in_kernel,
        grid=(N // RBI,),
        in_specs=[
            pl.BlockSpec((RBI, cin), lambda r: (r, 0)),
            pl.BlockSpec((cin, cout), lambda r: (0, 0)),
            pl.BlockSpec((cout, 128), lambda r: (0, 0)),
        ],
        out_specs=[
            pl.BlockSpec((RBI, cout), lambda r: (r, 0)),
            pl.BlockSpec((RBI, 128), lambda r: (r, 0)),
        ],
        out_shape=[
            jax.ShapeDtypeStruct((N, cout), F32),
            jax.ShapeDtypeStruct((N, 128), F32),
        ],
    )(x, p['W'], amat)
    abT = ab.T
    bias2 = p['bias'][None, :]
    out = pl.pallas_call(
        functools.partial(_gat_agg_kernel, heads=heads, ch=ch, relu=relu),
        grid=(N // RB,),
        in_specs=[
            pl.BlockSpec((1, RB, HID), lambda r: (li, r, 0)),
            pl.BlockSpec((1, N, HID), lambda r: (li, 0, 0)),
            pl.BlockSpec((RB, 128), lambda r: (r, 0)),
            pl.BlockSpec((N, cout), lambda r: (0, 0)),
            pl.BlockSpec((128, N), lambda r: (0, 0)),
            pl.BlockSpec((RB, 128), lambda r: (r, 0)),
            pl.BlockSpec((1, cout), lambda r: (0, 0)),
        ],
        out_specs=pl.BlockSpec((RB, cout), lambda r: (r, 0)),
        out_shape=jax.ShapeDtypeStruct((N, cout), F32),
    )(emb, emb, thr[li], h, abT, ab, bias2)
    return out


def _mm_bias_kernel(x_ref, w_ref, b_ref, o_ref):
    o_ref[...] = _dot(x_ref[...], w_ref[...]) + b_ref[...]


def _attn_kernel(q_ref, k_ref, v_ref, o_ref):
    q = q_ref[...]
    k = k_ref[...]
    v = v_ref[...]
    dh = HID // NHEAD
    outs = []
    for hd in range(NHEAD):
        sl = slice(hd * dh, (hd + 1) * dh)
        s = jax.lax.dot_general(q[:, sl], k[:, sl], (((1,), (1,)), ((), ())),
                                preferred_element_type=F32) * 0.125
        e = jnp.exp(s)
        ssum = jnp.sum(e, axis=1, keepdims=True)
        outs.append(_dot(e, v[:, sl]) / ssum)
    o_ref[...] = jnp.concatenate(outs, axis=1)


def _ln(x, g, b):
    m = jnp.mean(x, axis=-1, keepdims=True)
    v = jnp.mean((x - m) * (x - m), axis=-1, keepdims=True)
    return (x - m) / jnp.sqrt(v + 1e-5) * g + b


def _post_kernel(x_ref, o_ref, ow_ref, ob_ref, g1_ref, b1_ref, w1_ref,
                 bb1_ref, w2_ref, bb2_ref, g2_ref, b2_ref, win_ref, skw_ref,
                 skb_ref, out_ref, *, skip):
    x = x_ref[...]
    a = _dot(o_ref[...], ow_ref[...]) + ob_ref[...]
    x1 = _ln(x + a, g1_ref[...], b1_ref[...])
    f = jnp.maximum(_dot(x1, w1_ref[...]) + bb1_ref[...], 0.0)
    f = _dot(f, w2_ref[...]) + bb2_ref[...]
    x2 = _ln(x1 + f, g2_ref[...], b2_ref[...])
    if skip:
        x2 = x2 + _dot(win_ref[...], skw_ref[...]) + skb_ref[...]
    out_ref[...] = x2


def _trans_layer(x, p, window, skw, skb, skip):
    qkv = pl.pallas_call(
        _mm_bias_kernel,
        grid=(N // RBI,),
        in_specs=[
            pl.BlockSpec((RBI, HID), lambda r: (r, 0)),
            pl.BlockSpec((HID, 3 * HID), lambda r: (0, 0)),
            pl.BlockSpec((1, 3 * HID), lambda r: (0, 0)),
        ],
        out_specs=pl.BlockSpec((RBI, 3 * HID), lambda r: (r, 0)),
        out_shape=jax.ShapeDtypeStruct((N, 3 * HID), F32),
    )(x, p['in_w'].T, p['in_b'][None, :])
    o = pl.pallas_call(
        _attn_kernel,
        grid=(N // RBI,),
        in_specs=[
            pl.BlockSpec((RBI, HID), lambda r: (r, 0)),
            pl.BlockSpec((N, HID), lambda r: (0, 1)),
            pl.BlockSpec((N, HID), lambda r: (0, 2)),
        ],
        out_specs=pl.BlockSpec((RBI, HID), lambda r: (r, 0)),
        out_shape=jax.ShapeDtypeStruct((N, HID), F32),
    )(qkv, qkv, qkv)
    out = pl.pallas_call(
        functools.partial(_post_kernel, skip=skip),
        grid=(N // RBI,),
        in_specs=[
            pl.BlockSpec((RBI, HID), lambda r: (r, 0)),
            pl.BlockSpec((RBI, HID), lambda r: (r, 0)),
            pl.BlockSpec((HID, HID), lambda r: (0, 0)),
            pl.BlockSpec((1, HID), lambda r: (0, 0)),
            pl.BlockSpec((1, HID), lambda r: (0, 0)),
            pl.BlockSpec((1, HID), lambda r: (0, 0)),
            pl.BlockSpec((HID, DFF), lambda r: (0, 0)),
            pl.BlockSpec((1, DFF), lambda r: (0, 0)),
            pl.BlockSpec((DFF, HID), lambda r: (0, 0)),
            pl.BlockSpec((1, HID), lambda r: (0, 0)),
            pl.BlockSpec((1, HID), lambda r: (0, 0)),
            pl.BlockSpec((1, HID), lambda r: (0, 0)),
            pl.BlockSpec((RBI, IN_C), lambda r: (r, 0)),
            pl.BlockSpec((IN_C, HID), lambda r: (0, 0)),
            pl.BlockSpec((1, HID), lambda r: (0, 0)),
        ],
        out_specs=pl.BlockSpec((RBI, HID), lambda r: (r, 0)),
        out_shape=jax.ShapeDtypeStruct((N, HID), F32),
    )(x, o, p['out_w'].T, p['out_b'][None, :], p['ln1_g'][None, :],
      p['ln1_b'][None, :], p['l1_w'].T, p['l1_b'][None, :], p['l2_w'].T,
      p['l2_b'][None, :], p['ln2_g'][None, :], p['ln2_b'][None, :],
      window, skw, skb)
    return out


def kernel(window, params):
    x = window
    gl_w = jnp.stack(params['gl_W'])
    nl = gl_w.shape[0]
    emb = pl.pallas_call(
        _emb_kernel,
        grid=(nl,),
        in_specs=[
            pl.BlockSpec((N, IN_C), lambda l: (0, 0)),
            pl.BlockSpec((1, IN_C, HID), lambda l: (l, 0, 0)),
        ],
        out_specs=pl.BlockSpec((1, N, HID), lambda l: (l, 0, 0)),
        out_shape=jax.ShapeDtypeStruct((nl, N, HID), F32),
    )(x, gl_w)
    thr = pl.pallas_call(
        _thresh_kernel,
        grid=(nl, N // RB),
        in_specs=[
            pl.BlockSpec((1, RB, HID), lambda l, r: (l, r, 0)),
            pl.BlockSpec((1, N, HID), lambda l, r: (l, 0, 0)),
        ],
        out_specs=pl.BlockSpec((1, RB, 128), lambda l, r: (l, r, 0)),
        out_shape=jax.ShapeDtypeStruct((nl, N, 128), F32),
    )(emb, emb)

    h = x
    for i, p in enumerate(params['enc']):
        h = _gat_layer(h, emb, thr, i, p, HEADS, HID // HEADS, relu=True)

    skw = params['skip_w'].T
    skb = params['skip_b'][None, :]
    ht = h
    for li, p in enumerate(params['trans']):
        ht = _trans_layer(ht, p, window, skw, skb,
                          skip=(li == len(params['trans']) - 1))

    d = ht
    dec = params['dec']
    for i in range(len(dec) - 1):
        d = _gat_layer(d, emb, thr, 3 + i, dec[i], HEADS, HID // HEADS,
                       relu=True)
    d = _gat_layer(d, emb, thr, 3 + len(dec) - 1, dec[-1], 1, OUT_C,
                   relu=False)
    return d.T


# speedup vs baseline: 60.3814x; 1.0530x over previous
import functools

import jax
import jax.numpy as jnp
from jax.experimental import pallas as pl

N = 2048
IN_C = 128
HID = 256
OUT_C = 128
HEADS = 4
TOP_K = 16
NHEAD = 4
DFF = 512

RB = 256
RBI = 512

F32 = jnp.float32


def _dot(a, b):
    return jnp.dot(a, b, preferred_element_type=F32)


def _emb_kernel(x_ref, w_ref, out_ref):
    out_ref[0] = jnp.tanh(_dot(x_ref[...], w_ref[0]))


def _thresh_kernel(embr_ref, embf_ref, msk_ref):
    er = embr_ref[0]
    ef = embf_ref[0]
    sim = jax.lax.dot_general(er, ef, (((1,), (1,)), ((), ())),
                              preferred_element_type=F32)
    a = sim[:, 0:128]
    b = jnp.full((RB, 128), -jnp.inf, F32)
    c = b
    for ci in range(1, N // 128):
        x = sim[:, ci * 128:(ci + 1) * 128]
        lo = jnp.minimum(a, x)
        a = jnp.maximum(a, x)
        lo2 = jnp.minimum(b, lo)
        b = jnp.maximum(b, lo)
        c = jnp.maximum(c, lo2)
    w = jnp.concatenate([a, b, c], axis=1)
    m = None
    m_prev = None
    for k in range(TOP_K + 1):
        m_prev = m
        m = jnp.max(w, axis=1, keepdims=True)
        if k < TOP_K:
            w = jnp.where(w < m, w, -jnp.inf)
    t = (m_prev + m) * 0.5
    msk_ref[0] = (sim >= t).astype(jnp.int8)


def _gat_in_kernel(x_ref, w_ref, a_ref, h_ref, ab_ref):
    h = _dot(x_ref[...], w_ref[...])
    h_ref[...] = h
    ab_ref[...] = _dot(h, a_ref[...])


def _gat_agg_kernel(msk_ref, h_ref, abT_ref, ab_ref, b_ref, o_ref, *,
                    heads, ch, relu):
    i = pl.program_id(0)
    cols = jax.lax.broadcasted_iota(jnp.int32, (RB, N), 1)
    rowid = i * RB + jax.lax.broadcasted_iota(jnp.int32, (RB, 1), 0)
    B = msk_ref[0].astype(F32) + (cols == rowid).astype(F32)
    hf = h_ref[...]
    outs = []
    for hd in range(heads):
        asrc = abT_ref[hd:hd + 1, :]
        adst = ab_ref[:, 64 + hd:65 + hd]
        d = asrc + adst
        d = jnp.where(d >= 0, d, 0.2 * d)
        e = jnp.exp(d) * B
        ssum = jnp.sum(e, axis=1, keepdims=True)
        outs.append(_dot(e, hf[:, hd * ch:(hd + 1) * ch]) / (ssum + 1e-16))
    o = jnp.concatenate(outs, axis=1) if heads > 1 else outs[0]
    o = o + b_ref[...]
    if relu:
        o = jnp.maximum(o, 0.0)
    o_ref[...] = o


def _gat_layer(x, msk, li, p, heads, ch, relu):
    cin = x.shape[1]
    cout = heads * ch
    att_src = p['att_src']
    att_dst = p['att_dst']
    amat = jnp.zeros((cout, 128), F32)
    for hd in range(heads):
        amat = amat.at[hd * ch:(hd + 1) * ch, hd].set(att_src[hd])
        amat = amat.at[hd * ch:(hd + 1) * ch, 64 + hd].set(att_dst[hd])
    h, ab = pl.pallas_call(
        _gat_in_kernel,
        grid=(N // RBI,),
        in_specs=[
            pl.BlockSpec((RBI, cin), lambda r: (r, 0)),
            pl.BlockSpec((cin, cout), lambda r: (0, 0)),
            pl.BlockSpec((cout, 128), lambda r: (0, 0)),
        ],
        out_specs=[
            pl.BlockSpec((RBI, cout), lambda r: (r, 0)),
            pl.BlockSpec((RBI, 128), lambda r: (r, 0)),
        ],
        out_shape=[
            jax.ShapeDtypeStruct((N, cout), F32),
            jax.ShapeDtypeStruct((N, 128), F32),
        ],
    )(x, p['W'], amat)
    abT = ab.T
    bias2 = p['bias'][None, :]
    out = pl.pallas_call(
        functools.partial(_gat_agg_kernel, heads=heads, ch=ch, relu=relu),
        grid=(N // RB,),
        in_specs=[
            pl.BlockSpec((1, RB, N), lambda r: (li, r, 0)),
            pl.BlockSpec((N, cout), lambda r: (0, 0)),
            pl.BlockSpec((128, N), lambda r: (0, 0)),
            pl.BlockSpec((RB, 128), lambda r: (r, 0)),
            pl.BlockSpec((1, cout), lambda r: (0, 0)),
        ],
        out_specs=pl.BlockSpec((RB, cout), lambda r: (r, 0)),
        out_shape=jax.ShapeDtypeStruct((N, cout), F32),
    )(msk, h, abT, ab, bias2)
    return out


def _mm_bias_kernel(x_ref, w_ref, b_ref, o_ref):
    o_ref[...] = _dot(x_ref[...], w_ref[...]) + b_ref[...]


def _attn_kernel(q_ref, k_ref, v_ref, o_ref):
    q = q_ref[...]
    k = k_ref[...]
    v = v_ref[...]
    dh = HID // NHEAD
    outs = []
    for hd in range(NHEAD):
        sl = slice(hd * dh, (hd + 1) * dh)
        s = jax.lax.dot_general(q[:, sl], k[:, sl], (((1,), (1,)), ((), ())),
                                preferred_element_type=F32) * 0.125
        e = jnp.exp(s)
        ssum = jnp.sum(e, axis=1, keepdims=True)
        outs.append(_dot(e, v[:, sl]) / ssum)
    o_ref[...] = jnp.concatenate(outs, axis=1)


def _ln(x, g, b):
    m = jnp.mean(x, axis=-1, keepdims=True)
    v = jnp.mean((x - m) * (x - m), axis=-1, keepdims=True)
    return (x - m) / jnp.sqrt(v + 1e-5) * g + b


def _post_kernel(x_ref, o_ref, ow_ref, ob_ref, g1_ref, b1_ref, w1_ref,
                 bb1_ref, w2_ref, bb2_ref, g2_ref, b2_ref, win_ref, skw_ref,
                 skb_ref, out_ref, *, skip):
    x = x_ref[...]
    a = _dot(o_ref[...], ow_ref[...]) + ob_ref[...]
    x1 = _ln(x + a, g1_ref[...], b1_ref[...])
    f = jnp.maximum(_dot(x1, w1_ref[...]) + bb1_ref[...], 0.0)
    f = _dot(f, w2_ref[...]) + bb2_ref[...]
    x2 = _ln(x1 + f, g2_ref[...], b2_ref[...])
    if skip:
        x2 = x2 + _dot(win_ref[...], skw_ref[...]) + skb_ref[...]
    out_ref[...] = x2


def _trans_layer(x, p, window, skw, skb, skip):
    qkv = pl.pallas_call(
        _mm_bias_kernel,
        grid=(N // RBI,),
        in_specs=[
            pl.BlockSpec((RBI, HID), lambda r: (r, 0)),
            pl.BlockSpec((HID, 3 * HID), lambda r: (0, 0)),
            pl.BlockSpec((1, 3 * HID), lambda r: (0, 0)),
        ],
        out_specs=pl.BlockSpec((RBI, 3 * HID), lambda r: (r, 0)),
        out_shape=jax.ShapeDtypeStruct((N, 3 * HID), F32),
    )(x, p['in_w'].T, p['in_b'][None, :])
    o = pl.pallas_call(
        _attn_kernel,
        grid=(N // RBI,),
        in_specs=[
            pl.BlockSpec((RBI, HID), lambda r: (r, 0)),
            pl.BlockSpec((N, HID), lambda r: (0, 1)),
            pl.BlockSpec((N, HID), lambda r: (0, 2)),
        ],
        out_specs=pl.BlockSpec((RBI, HID), lambda r: (r, 0)),
        out_shape=jax.ShapeDtypeStruct((N, HID), F32),
    )(qkv, qkv, qkv)
    out = pl.pallas_call(
        functools.partial(_post_kernel, skip=skip),
        grid=(N // RBI,),
        in_specs=[
            pl.BlockSpec((RBI, HID), lambda r: (r, 0)),
            pl.BlockSpec((RBI, HID), lambda r: (r, 0)),
            pl.BlockSpec((HID, HID), lambda r: (0, 0)),
            pl.BlockSpec((1, HID), lambda r: (0, 0)),
            pl.BlockSpec((1, HID), lambda r: (0, 0)),
            pl.BlockSpec((1, HID), lambda r: (0, 0)),
            pl.BlockSpec((HID, DFF), lambda r: (0, 0)),
            pl.BlockSpec((1, DFF), lambda r: (0, 0)),
            pl.BlockSpec((DFF, HID), lambda r: (0, 0)),
            pl.BlockSpec((1, HID), lambda r: (0, 0)),
            pl.BlockSpec((1, HID), lambda r: (0, 0)),
            pl.BlockSpec((1, HID), lambda r: (0, 0)),
            pl.BlockSpec((RBI, IN_C), lambda r: (r, 0)),
            pl.BlockSpec((IN_C, HID), lambda r: (0, 0)),
            pl.BlockSpec((1, HID), lambda r: (0, 0)),
        ],
        out_specs=pl.BlockSpec((RBI, HID), lambda r: (r, 0)),
        out_shape=jax.ShapeDtypeStruct((N, HID), F32),
    )(x, o, p['out_w'].T, p['out_b'][None, :], p['ln1_g'][None, :],
      p['ln1_b'][None, :], p['l1_w'].T, p['l1_b'][None, :], p['l2_w'].T,
      p['l2_b'][None, :], p['ln2_g'][None, :], p['ln2_b'][None, :],
      window, skw, skb)
    return out


def kernel(window, params):
    x = window
    gl_w = jnp.stack(params['gl_W'])
    nl = gl_w.shape[0]
    emb = pl.pallas_call(
        _emb_kernel,
        grid=(nl,),
        in_specs=[
            pl.BlockSpec((N, IN_C), lambda l: (0, 0)),
            pl.BlockSpec((1, IN_C, HID), lambda l: (l, 0, 0)),
        ],
        out_specs=pl.BlockSpec((1, N, HID), lambda l: (l, 0, 0)),
        out_shape=jax.ShapeDtypeStruct((nl, N, HID), F32),
    )(x, gl_w)
    msk = pl.pallas_call(
        _thresh_kernel,
        grid=(nl, N // RB),
        in_specs=[
            pl.BlockSpec((1, RB, HID), lambda l, r: (l, r, 0)),
            pl.BlockSpec((1, N, HID), lambda l, r: (l, 0, 0)),
        ],
        out_specs=pl.BlockSpec((1, RB, N), lambda l, r: (l, r, 0)),
        out_shape=jax.ShapeDtypeStruct((nl, N, N), jnp.int8),
    )(emb, emb)

    h = x
    for i, p in enumerate(params['enc']):
        h = _gat_layer(h, msk, i, p, HEADS, HID // HEADS, relu=True)

    skw = params['skip_w'].T
    skb = params['skip_b'][None, :]
    ht = h
    for li, p in enumerate(params['trans']):
        ht = _trans_layer(ht, p, window, skw, skb,
                          skip=(li == len(params['trans']) - 1))

    d = ht
    dec = params['dec']
    for i in range(len(dec) - 1):
        d = _gat_layer(d, msk, 3 + i, dec[i], HEADS, HID // HEADS, relu=True)
    d = _gat_layer(d, msk, 3 + len(dec) - 1, dec[-1], 1, OUT_C, relu=False)
    return d.T


# speedup vs baseline: 61.1155x; 1.0122x over previous
import functools

import jax
import jax.numpy as jnp
from jax.experimental import pallas as pl

N = 2048
IN_C = 128
HID = 256
OUT_C = 128
HEADS = 4
TOP_K = 16
NHEAD = 4
DFF = 512

RB = 256
RBI = 512

F32 = jnp.float32


def _dot(a, b):
    return jnp.dot(a, b, preferred_element_type=F32)


def _emb_kernel(x_ref, w_ref, out_ref):
    out_ref[0] = jnp.tanh(_dot(x_ref[...], w_ref[0]))


def _thresh_kernel(embr_ref, embf_ref, msk_ref):
    er = embr_ref[0]
    ef = embf_ref[0]
    sim = jax.lax.dot_general(er, ef, (((1,), (1,)), ((), ())),
                              preferred_element_type=F32)
    a = sim[:, 0:128]
    b = jnp.full((RB, 128), -jnp.inf, F32)
    c = b
    for ci in range(1, N // 128):
        x = sim[:, ci * 128:(ci + 1) * 128]
        lo = jnp.minimum(a, x)
        a = jnp.maximum(a, x)
        lo2 = jnp.minimum(b, lo)
        b = jnp.maximum(b, lo)
        c = jnp.maximum(c, lo2)
    w = jnp.concatenate([a, b, c], axis=1)
    m = None
    m_prev = None
    for k in range(TOP_K + 1):
        m_prev = m
        m = jnp.max(w, axis=1, keepdims=True)
        if k < TOP_K:
            w = jnp.where(w < m, w, -jnp.inf)
    t = (m_prev + m) * 0.5
    msk_ref[0] = (sim >= t).astype(jnp.int8)


def _gat_in_kernel(x_ref, w_ref, a_ref, h_ref, ab_ref):
    h = _dot(x_ref[...], w_ref[...])
    h_ref[...] = h
    ab_ref[...] = _dot(h, a_ref[...])


def _gat_agg_kernel(msk_ref, h_ref, abT_ref, ab_ref, b_ref, o_ref, *,
                    heads, ch, relu):
    i = pl.program_id(0)
    cols = jax.lax.broadcasted_iota(jnp.int32, (RB, N), 1)
    rowid = i * RB + jax.lax.broadcasted_iota(jnp.int32, (RB, 1), 0)
    B = msk_ref[0].astype(F32) + (cols == rowid).astype(F32)
    hf = h_ref[...]
    outs = []
    for hd in range(heads):
        asrc = abT_ref[hd:hd + 1, :]
        adst = ab_ref[:, 64 + hd:65 + hd]
        ea, ea5 = jnp.exp(asrc), jnp.exp(asrc * 0.2)
        eb, eb5 = jnp.exp(adst), jnp.exp(adst * 0.2)
        e = jnp.where(asrc >= -adst, ea * eb, ea5 * eb5) * B
        ssum = jnp.sum(e, axis=1, keepdims=True)
        outs.append(_dot(e, hf[:, hd * ch:(hd + 1) * ch]) / (ssum + 1e-16))
    o = jnp.concatenate(outs, axis=1) if heads > 1 else outs[0]
    o = o + b_ref[...]
    if relu:
        o = jnp.maximum(o, 0.0)
    o_ref[...] = o


def _gat_layer(x, msk, li, p, heads, ch, relu):
    cin = x.shape[1]
    cout = heads * ch
    att_src = p['att_src']
    att_dst = p['att_dst']
    amat = jnp.zeros((cout, 128), F32)
    for hd in range(heads):
        amat = amat.at[hd * ch:(hd + 1) * ch, hd].set(att_src[hd])
        amat = amat.at[hd * ch:(hd + 1) * ch, 64 + hd].set(att_dst[hd])
    h, ab = pl.pallas_call(
        _gat_in_kernel,
        grid=(N // RBI,),
        in_specs=[
            pl.BlockSpec((RBI, cin), lambda r: (r, 0)),
            pl.BlockSpec((cin, cout), lambda r: (0, 0)),
            pl.BlockSpec((cout, 128), lambda r: (0, 0)),
        ],
        out_specs=[
            pl.BlockSpec((RBI, cout), lambda r: (r, 0)),
            pl.BlockSpec((RBI, 128), lambda r: (r, 0)),
        ],
        out_shape=[
            jax.ShapeDtypeStruct((N, cout), F32),
            jax.ShapeDtypeStruct((N, 128), F32),
        ],
    )(x, p['W'], amat)
    abT = ab.T
    bias2 = p['bias'][None, :]
    out = pl.pallas_call(
        functools.partial(_gat_agg_kernel, heads=heads, ch=ch, relu=relu),
        grid=(N // RB,),
        in_specs=[
            pl.BlockSpec((1, RB, N), lambda r: (li, r, 0)),
            pl.BlockSpec((N, cout), lambda r: (0, 0)),
            pl.BlockSpec((128, N), lambda r: (0, 0)),
            pl.BlockSpec((RB, 128), lambda r: (r, 0)),
            pl.BlockSpec((1, cout), lambda r: (0, 0)),
        ],
        out_specs=pl.BlockSpec((RB, cout), lambda r: (r, 0)),
        out_shape=jax.ShapeDtypeStruct((N, cout), F32),
    )(msk, h, abT, ab, bias2)
    return out


def _mm_bias_kernel(x_ref, w_ref, b_ref, o_ref):
    o_ref[...] = _dot(x_ref[...], w_ref[...]) + b_ref[...]


def _attn_kernel(q_ref, k_ref, v_ref, o_ref):
    q = q_ref[...]
    k = k_ref[...]
    v = v_ref[...]
    dh = HID // NHEAD
    outs = []
    for hd in range(NHEAD):
        sl = slice(hd * dh, (hd + 1) * dh)
        s = jax.lax.dot_general(q[:, sl], k[:, sl], (((1,), (1,)), ((), ())),
                                preferred_element_type=F32) * 0.125
        e = jnp.exp(s)
        ssum = jnp.sum(e, axis=1, keepdims=True)
        outs.append(_dot(e, v[:, sl]) / ssum)
    o_ref[...] = jnp.concatenate(outs, axis=1)


def _ln(x, g, b):
    m = jnp.mean(x, axis=-1, keepdims=True)
    v = jnp.mean((x - m) * (x - m), axis=-1, keepdims=True)
    return (x - m) / jnp.sqrt(v + 1e-5) * g + b


def _post_kernel(x_ref, o_ref, ow_ref, ob_ref, g1_ref, b1_ref, w1_ref,
                 bb1_ref, w2_ref, bb2_ref, g2_ref, b2_ref, win_ref, skw_ref,
                 skb_ref, out_ref, *, skip):
    x = x_ref[...]
    a = _dot(o_ref[...], ow_ref[...]) + ob_ref[...]
    x1 = _ln(x + a, g1_ref[...], b1_ref[...])
    f = jnp.maximum(_dot(x1, w1_ref[...]) + bb1_ref[...], 0.0)
    f = _dot(f, w2_ref[...]) + bb2_ref[...]
    x2 = _ln(x1 + f, g2_ref[...], b2_ref[...])
    if skip:
        x2 = x2 + _dot(win_ref[...], skw_ref[...]) + skb_ref[...]
    out_ref[...] = x2


def _trans_layer(x, p, window, skw, skb, skip):
    qkv = pl.pallas_call(
        _mm_bias_kernel,
        grid=(N // RBI,),
        in_specs=[
            pl.BlockSpec((RBI, HID), lambda r: (r, 0)),
            pl.BlockSpec((HID, 3 * HID), lambda r: (0, 0)),
            pl.BlockSpec((1, 3 * HID), lambda r: (0, 0)),
        ],
        out_specs=pl.BlockSpec((RBI, 3 * HID), lambda r: (r, 0)),
        out_shape=jax.ShapeDtypeStruct((N, 3 * HID), F32),
    )(x, p['in_w'].T, p['in_b'][None, :])
    o = pl.pallas_call(
        _attn_kernel,
        grid=(N // RBI,),
        in_specs=[
            pl.BlockSpec((RBI, HID), lambda r: (r, 0)),
            pl.BlockSpec((N, HID), lambda r: (0, 1)),
            pl.BlockSpec((N, HID), lambda r: (0, 2)),
        ],
        out_specs=pl.BlockSpec((RBI, HID), lambda r: (r, 0)),
        out_shape=jax.ShapeDtypeStruct((N, HID), F32),
    )(qkv, qkv, qkv)
    out = pl.pallas_call(
        functools.partial(_post_kernel, skip=skip),
        grid=(N // RBI,),
        in_specs=[
            pl.BlockSpec((RBI, HID), lambda r: (r, 0)),
            pl.BlockSpec((RBI, HID), lambda r: (r, 0)),
            pl.BlockSpec((HID, HID), lambda r: (0, 0)),
            pl.BlockSpec((1, HID), lambda r: (0, 0)),
            pl.BlockSpec((1, HID), lambda r: (0, 0)),
            pl.BlockSpec((1, HID), lambda r: (0, 0)),
            pl.BlockSpec((HID, DFF), lambda r: (0, 0)),
            pl.BlockSpec((1, DFF), lambda r: (0, 0)),
            pl.BlockSpec((DFF, HID), lambda r: (0, 0)),
            pl.BlockSpec((1, HID), lambda r: (0, 0)),
            pl.BlockSpec((1, HID), lambda r: (0, 0)),
            pl.BlockSpec((1, HID), lambda r: (0, 0)),
            pl.BlockSpec((RBI, IN_C), lambda r: (r, 0)),
            pl.BlockSpec((IN_C, HID), lambda r: (0, 0)),
            pl.BlockSpec((1, HID), lambda r: (0, 0)),
        ],
        out_specs=pl.BlockSpec((RBI, HID), lambda r: (r, 0)),
        out_shape=jax.ShapeDtypeStruct((N, HID), F32),
    )(x, o, p['out_w'].T, p['out_b'][None, :], p['ln1_g'][None, :],
      p['ln1_b'][None, :], p['l1_w'].T, p['l1_b'][None, :], p['l2_w'].T,
      p['l2_b'][None, :], p['ln2_g'][None, :], p['ln2_b'][None, :],
      window, skw, skb)
    return out


def kernel(window, params):
    x = window
    gl_w = jnp.stack(params['gl_W'])
    nl = gl_w.shape[0]
    emb = pl.pallas_call(
        _emb_kernel,
        grid=(nl,),
        in_specs=[
            pl.BlockSpec((N, IN_C), lambda l: (0, 0)),
            pl.BlockSpec((1, IN_C, HID), lambda l: (l, 0, 0)),
        ],
        out_specs=pl.BlockSpec((1, N, HID), lambda l: (l, 0, 0)),
        out_shape=jax.ShapeDtypeStruct((nl, N, HID), F32),
    )(x, gl_w)
    msk = pl.pallas_call(
        _thresh_kernel,
        grid=(nl, N // RB),
        in_specs=[
            pl.BlockSpec((1, RB, HID), lambda l, r: (l, r, 0)),
            pl.BlockSpec((1, N, HID), lambda l, r: (l, 0, 0)),
        ],
        out_specs=pl.BlockSpec((1, RB, N), lambda l, r: (l, r, 0)),
        out_shape=jax.ShapeDtypeStruct((nl, N, N), jnp.int8),
    )(emb, emb)

    h = x
    for i, p in enumerate(params['enc']):
        h = _gat_layer(h, msk, i, p, HEADS, HID // HEADS, relu=True)

    skw = params['skip_w'].T
    skb = params['skip_b'][None, :]
    ht = h
    for li, p in enumerate(params['trans']):
        ht = _trans_layer(ht, p, window, skw, skb,
                          skip=(li == len(params['trans']) - 1))

    d = ht
    dec = params['dec']
    for i in range(len(dec) - 1):
        d = _gat_layer(d, msk, 3 + i, dec[i], HEADS, HID // HEADS, relu=True)
    d = _gat_layer(d, msk, 3 + len(dec) - 1, dec[-1], 1, OUT_C, relu=False)
    return d.T


# speedup vs baseline: 63.0443x; 1.0316x over previous
import functools

import jax
import jax.numpy as jnp
from jax.experimental import pallas as pl

N = 2048
IN_C = 128
HID = 256
OUT_C = 128
HEADS = 4
TOP_K = 16
NHEAD = 4
DFF = 512

RB = 256
RBI = 512

F32 = jnp.float32


def _dot(a, b):
    return jnp.dot(a, b, preferred_element_type=F32)


def _emb_kernel(x_ref, w_ref, out_ref):
    out_ref[0] = jnp.tanh(_dot(x_ref[...], w_ref[0]))


def _thresh_kernel(embr_ref, embf_ref, msk_ref):
    er = embr_ref[0]
    ef = embf_ref[0]
    sim = jax.lax.dot_general(er, ef, (((1,), (1,)), ((), ())),
                              preferred_element_type=F32)
    a = sim[:, 0:128]
    b = jnp.full((RB, 128), -jnp.inf, F32)
    c = b
    for ci in range(1, N // 128):
        x = sim[:, ci * 128:(ci + 1) * 128]
        lo = jnp.minimum(a, x)
        a = jnp.maximum(a, x)
        lo2 = jnp.minimum(b, lo)
        b = jnp.maximum(b, lo)
        c = jnp.maximum(c, lo2)
    w = jnp.concatenate([a, b, c], axis=1)
    m = None
    m_prev = None
    for k in range(TOP_K + 1):
        m_prev = m
        m = jnp.max(w, axis=1, keepdims=True)
        if k < TOP_K:
            w = jnp.where(w < m, w, -jnp.inf)
    t = (m_prev + m) * 0.5
    msk_ref[0] = (sim >= t).astype(jnp.int8)


def _gat_in_kernel(x_ref, w_ref, a_ref, h_ref, ab_ref, abT_ref):
    h = _dot(x_ref[...], w_ref[...])
    h_ref[...] = h
    ab = _dot(h, a_ref[...])
    ab_ref[...] = ab
    abT_ref[...] = ab.T


def _gat_agg_kernel(msk_ref, h_ref, abT_ref, ab_ref, b_ref, o_ref, *,
                    heads, ch, relu, t_out=False):
    i = pl.program_id(0)
    cols = jax.lax.broadcasted_iota(jnp.int32, (RB, N), 1)
    rowid = i * RB + jax.lax.broadcasted_iota(jnp.int32, (RB, 1), 0)
    B = msk_ref[0].astype(F32) + (cols == rowid).astype(F32)
    hf = h_ref[...]
    outs = []
    for hd in range(heads):
        asrc = abT_ref[hd:hd + 1, :]
        adst = ab_ref[:, 64 + hd:65 + hd]
        ea, ea5 = jnp.exp(asrc), jnp.exp(asrc * 0.2)
        eb, eb5 = jnp.exp(adst), jnp.exp(adst * 0.2)
        e = jnp.where(asrc >= -adst, ea * eb, ea5 * eb5) * B
        ssum = jnp.sum(e, axis=1, keepdims=True)
        outs.append(_dot(e, hf[:, hd * ch:(hd + 1) * ch]) / (ssum + 1e-16))
    o = jnp.concatenate(outs, axis=1) if heads > 1 else outs[0]
    o = o + b_ref[...]
    if relu:
        o = jnp.maximum(o, 0.0)
    if t_out:
        o_ref[...] = o.T
    else:
        o_ref[...] = o


def _gat_layer(x, msk, li, p, heads, ch, relu, t_out=False):
    cin = x.shape[1]
    cout = heads * ch
    att_src = p['att_src']
    att_dst = p['att_dst']
    amat = jnp.zeros((cout, 128), F32)
    for hd in range(heads):
        amat = amat.at[hd * ch:(hd + 1) * ch, hd].set(att_src[hd])
        amat = amat.at[hd * ch:(hd + 1) * ch, 64 + hd].set(att_dst[hd])
    h, ab, abT = pl.pallas_call(
        _gat_in_kernel,
        grid=(N // RBI,),
        in_specs=[
            pl.BlockSpec((RBI, cin), lambda r: (r, 0)),
            pl.BlockSpec((cin, cout), lambda r: (0, 0)),
            pl.BlockSpec((cout, 128), lambda r: (0, 0)),
        ],
        out_specs=[
            pl.BlockSpec((RBI, cout), lambda r: (r, 0)),
            pl.BlockSpec((RBI, 128), lambda r: (r, 0)),
            pl.BlockSpec((128, RBI), lambda r: (0, r)),
        ],
        out_shape=[
            jax.ShapeDtypeStruct((N, cout), F32),
            jax.ShapeDtypeStruct((N, 128), F32),
            jax.ShapeDtypeStruct((128, N), F32),
        ],
    )(x, p['W'], amat)
    bias2 = p['bias'][None, :]
    if t_out:
        ospec = pl.BlockSpec((cout, RB), lambda r: (0, r))
        oshape = jax.ShapeDtypeStruct((cout, N), F32)
    else:
        ospec = pl.BlockSpec((RB, cout), lambda r: (r, 0))
        oshape = jax.ShapeDtypeStruct((N, cout), F32)
    out = pl.pallas_call(
        functools.partial(_gat_agg_kernel, heads=heads, ch=ch, relu=relu,
                          t_out=t_out),
        grid=(N // RB,),
        in_specs=[
            pl.BlockSpec((1, RB, N), lambda r: (li, r, 0)),
            pl.BlockSpec((N, cout), lambda r: (0, 0)),
            pl.BlockSpec((128, N), lambda r: (0, 0)),
            pl.BlockSpec((RB, 128), lambda r: (r, 0)),
            pl.BlockSpec((1, cout), lambda r: (0, 0)),
        ],
        out_specs=ospec,
        out_shape=oshape,
    )(msk, h, abT, ab, bias2)
    return out


def _mm_bias_kernel(x_ref, w_ref, b_ref, o_ref):
    o_ref[...] = _dot(x_ref[...], w_ref[...]) + b_ref[...]


def _attn_kernel(q_ref, k_ref, v_ref, o_ref):
    q = q_ref[...]
    k = k_ref[...]
    v = v_ref[...]
    dh = HID // NHEAD
    outs = []
    for hd in range(NHEAD):
        sl = slice(hd * dh, (hd + 1) * dh)
        s = jax.lax.dot_general(q[:, sl], k[:, sl], (((1,), (1,)), ((), ())),
                                preferred_element_type=F32) * 0.125
        e = jnp.exp(s)
        ssum = jnp.sum(e, axis=1, keepdims=True)
        outs.append(_dot(e.astype(jnp.bfloat16),
                         v[:, sl].astype(jnp.bfloat16)) / ssum)
    o_ref[...] = jnp.concatenate(outs, axis=1)


def _ln(x, g, b):
    m = jnp.mean(x, axis=-1, keepdims=True)
    v = jnp.mean((x - m) * (x - m), axis=-1, keepdims=True)
    return (x - m) / jnp.sqrt(v + 1e-5) * g + b


def _post_kernel(x_ref, o_ref, ow_ref, ob_ref, g1_ref, b1_ref, w1_ref,
                 bb1_ref, w2_ref, bb2_ref, g2_ref, b2_ref, win_ref, skw_ref,
                 skb_ref, out_ref, *, skip):
    x = x_ref[...]
    a = _dot(o_ref[...], ow_ref[...]) + ob_ref[...]
    x1 = _ln(x + a, g1_ref[...], b1_ref[...])
    f = jnp.maximum(_dot(x1, w1_ref[...]) + bb1_ref[...], 0.0)
    f = _dot(f, w2_ref[...]) + bb2_ref[...]
    x2 = _ln(x1 + f, g2_ref[...], b2_ref[...])
    if skip:
        x2 = x2 + _dot(win_ref[...], skw_ref[...]) + skb_ref[...]
    out_ref[...] = x2


def _trans_layer(x, p, window, skw, skb, skip):
    qkv = pl.pallas_call(
        _mm_bias_kernel,
        grid=(N // RBI,),
        in_specs=[
            pl.BlockSpec((RBI, HID), lambda r: (r, 0)),
            pl.BlockSpec((HID, 3 * HID), lambda r: (0, 0)),
            pl.BlockSpec((1, 3 * HID), lambda r: (0, 0)),
        ],
        out_specs=pl.BlockSpec((RBI, 3 * HID), lambda r: (r, 0)),
        out_shape=jax.ShapeDtypeStruct((N, 3 * HID), F32),
    )(x, p['in_w'].T, p['in_b'][None, :])
    o = pl.pallas_call(
        _attn_kernel,
        grid=(N // RBI,),
        in_specs=[
            pl.BlockSpec((RBI, HID), lambda r: (r, 0)),
            pl.BlockSpec((N, HID), lambda r: (0, 1)),
            pl.BlockSpec((N, HID), lambda r: (0, 2)),
        ],
        out_specs=pl.BlockSpec((RBI, HID), lambda r: (r, 0)),
        out_shape=jax.ShapeDtypeStruct((N, HID), F32),
    )(qkv, qkv, qkv)
    out = pl.pallas_call(
        functools.partial(_post_kernel, skip=skip),
        grid=(N // RBI,),
        in_specs=[
            pl.BlockSpec((RBI, HID), lambda r: (r, 0)),
            pl.BlockSpec((RBI, HID), lambda r: (r, 0)),
            pl.BlockSpec((HID, HID), lambda r: (0, 0)),
            pl.BlockSpec((1, HID), lambda r: (0, 0)),
            pl.BlockSpec((1, HID), lambda r: (0, 0)),
            pl.BlockSpec((1, HID), lambda r: (0, 0)),
            pl.BlockSpec((HID, DFF), lambda r: (0, 0)),
            pl.BlockSpec((1, DFF), lambda r: (0, 0)),
            pl.BlockSpec((DFF, HID), lambda r: (0, 0)),
            pl.BlockSpec((1, HID), lambda r: (0, 0)),
            pl.BlockSpec((1, HID), lambda r: (0, 0)),
            pl.BlockSpec((1, HID), lambda r: (0, 0)),
            pl.BlockSpec((RBI, IN_C), lambda r: (r, 0)),
            pl.BlockSpec((IN_C, HID), lambda r: (0, 0)),
            pl.BlockSpec((1, HID), lambda r: (0, 0)),
        ],
        out_specs=pl.BlockSpec((RBI, HID), lambda r: (r, 0)),
        out_shape=jax.ShapeDtypeStruct((N, HID), F32),
    )(x, o, p['out_w'].T, p['out_b'][None, :], p['ln1_g'][None, :],
      p['ln1_b'][None, :], p['l1_w'].T, p['l1_b'][None, :], p['l2_w'].T,
      p['l2_b'][None, :], p['ln2_g'][None, :], p['ln2_b'][None, :],
      window, skw, skb)
    return out


def kernel(window, params):
    x = window
    gl_w = jnp.stack(params['gl_W'])
    nl = gl_w.shape[0]
    emb = pl.pallas_call(
        _emb_kernel,
        grid=(nl,),
        in_specs=[
            pl.BlockSpec((N, IN_C), lambda l: (0, 0)),
            pl.BlockSpec((1, IN_C, HID), lambda l: (l, 0, 0)),
        ],
        out_specs=pl.BlockSpec((1, N, HID), lambda l: (l, 0, 0)),
        out_shape=jax.ShapeDtypeStruct((nl, N, HID), F32),
    )(x, gl_w)
    msk = pl.pallas_call(
        _thresh_kernel,
        grid=(nl, N // RB),
        in_specs=[
            pl.BlockSpec((1, RB, HID), lambda l, r: (l, r, 0)),
            pl.BlockSpec((1, N, HID), lambda l, r: (l, 0, 0)),
        ],
        out_specs=pl.BlockSpec((1, RB, N), lambda l, r: (l, r, 0)),
        out_shape=jax.ShapeDtypeStruct((nl, N, N), jnp.int8),
    )(emb, emb)

    h = x
    for i, p in enumerate(params['enc']):
        h = _gat_layer(h, msk, i, p, HEADS, HID // HEADS, relu=True)

    skw = params['skip_w'].T
    skb = params['skip_b'][None, :]
    ht = h
    for li, p in enumerate(params['trans']):
        ht = _trans_layer(ht, p, window, skw, skb,
                          skip=(li == len(params['trans']) - 1))

    d = ht
    dec = params['dec']
    for i in range(len(dec) - 1):
        d = _gat_layer(d, msk, 3 + i, dec[i], HEADS, HID // HEADS, relu=True)
    return _gat_layer(d, msk, 3 + len(dec) - 1, dec[-1], 1, OUT_C,
                      relu=False, t_out=True)


# speedup vs baseline: 70.5242x; 1.1186x over previous
import functools

import jax
import jax.numpy as jnp
from jax.experimental import pallas as pl

N = 2048
IN_C = 128
HID = 256
OUT_C = 128
HEADS = 4
TOP_K = 16
NHEAD = 4
DFF = 512

RB = 512
RBI = 512

F32 = jnp.float32


def _dot(a, b):
    return jnp.dot(a, b, preferred_element_type=F32)


def _emb_kernel(x_ref, w_ref, out_ref):
    out_ref[0] = jnp.tanh(_dot(x_ref[...], w_ref[0]))


def _thresh_kernel(embr_ref, embf_ref, msk_ref):
    er = embr_ref[0]
    ef = embf_ref[0]
    sim = jax.lax.dot_general(er, ef, (((1,), (1,)), ((), ())),
                              preferred_element_type=F32)
    a = sim[:, 0:128]
    b = jnp.full((RB, 128), -jnp.inf, F32)
    c = b
    for ci in range(1, N // 128):
        x = sim[:, ci * 128:(ci + 1) * 128]
        lo = jnp.minimum(a, x)
        a = jnp.maximum(a, x)
        lo2 = jnp.minimum(b, lo)
        b = jnp.maximum(b, lo)
        c = jnp.maximum(c, lo2)
    w = jnp.concatenate([a, b, c], axis=1)
    m = None
    m_prev = None
    for k in range(TOP_K + 1):
        m_prev = m
        m = jnp.max(w, axis=1, keepdims=True)
        if k < TOP_K:
            w = jnp.where(w < m, w, -jnp.inf)
    t = (m_prev + m) * 0.5
    msk_ref[0] = (sim >= t).astype(jnp.int8)


def _gat_in_kernel(x_ref, w_ref, a_ref, h_ref, ab_ref, abT_ref):
    h = _dot(x_ref[...], w_ref[...])
    h_ref[...] = h
    ab = _dot(h, a_ref[...])
    ab_ref[...] = ab
    abT_ref[...] = ab.T


def _gat_agg_kernel(msk_ref, h_ref, abT_ref, ab_ref, b_ref, o_ref, *,
                    heads, ch, relu, t_out=False):
    i = pl.program_id(0)
    cols = jax.lax.broadcasted_iota(jnp.int32, (RB, N), 1)
    rowid = i * RB + jax.lax.broadcasted_iota(jnp.int32, (RB, 1), 0)
    B = msk_ref[0].astype(F32) + (cols == rowid).astype(F32)
    hf = h_ref[...]
    outs = []
    for hd in range(heads):
        asrc = abT_ref[hd:hd + 1, :]
        adst = ab_ref[:, 64 + hd:65 + hd]
        ea, ea5 = jnp.exp(asrc), jnp.exp(asrc * 0.2)
        eb, eb5 = jnp.exp(adst), jnp.exp(adst * 0.2)
        e = jnp.where(asrc >= -adst, ea * eb, ea5 * eb5) * B
        ssum = jnp.sum(e, axis=1, keepdims=True)
        outs.append(_dot(e, hf[:, hd * ch:(hd + 1) * ch]) / (ssum + 1e-16))
    o = jnp.concatenate(outs, axis=1) if heads > 1 else outs[0]
    o = o + b_ref[...]
    if relu:
        o = jnp.maximum(o, 0.0)
    if t_out:
        o_ref[...] = o.T
    else:
        o_ref[...] = o


def _gat_layer(x, msk, li, p, heads, ch, relu, t_out=False):
    cin = x.shape[1]
    cout = heads * ch
    att_src = p['att_src']
    att_dst = p['att_dst']
    amat = jnp.zeros((cout, 128), F32)
    for hd in range(heads):
        amat = amat.at[hd * ch:(hd + 1) * ch, hd].set(att_src[hd])
        amat = amat.at[hd * ch:(hd + 1) * ch, 64 + hd].set(att_dst[hd])
    h, ab, abT = pl.pallas_call(
        _gat_in_kernel,
        grid=(N // RBI,),
        in_specs=[
            pl.BlockSpec((RBI, cin), lambda r: (r, 0)),
            pl.BlockSpec((cin, cout), lambda r: (0, 0)),
            pl.BlockSpec((cout, 128), lambda r: (0, 0)),
        ],
        out_specs=[
            pl.BlockSpec((RBI, cout), lambda r: (r, 0)),
            pl.BlockSpec((RBI, 128), lambda r: (r, 0)),
            pl.BlockSpec((128, RBI), lambda r: (0, r)),
        ],
        out_shape=[
            jax.ShapeDtypeStruct((N, cout), F32),
            jax.ShapeDtypeStruct((N, 128), F32),
            jax.ShapeDtypeStruct((128, N), F32),
        ],
    )(x, p['W'], amat)
    bias2 = p['bias'][None, :]
    if t_out:
        ospec = pl.BlockSpec((cout, RB), lambda r: (0, r))
        oshape = jax.ShapeDtypeStruct((cout, N), F32)
    else:
        ospec = pl.BlockSpec((RB, cout), lambda r: (r, 0))
        oshape = jax.ShapeDtypeStruct((N, cout), F32)
    out = pl.pallas_call(
        functools.partial(_gat_agg_kernel, heads=heads, ch=ch, relu=relu,
                          t_out=t_out),
        grid=(N // RB,),
        in_specs=[
            pl.BlockSpec((1, RB, N), lambda r: (li, r, 0)),
            pl.BlockSpec((N, cout), lambda r: (0, 0)),
            pl.BlockSpec((128, N), lambda r: (0, 0)),
            pl.BlockSpec((RB, 128), lambda r: (r, 0)),
            pl.BlockSpec((1, cout), lambda r: (0, 0)),
        ],
        out_specs=ospec,
        out_shape=oshape,
    )(msk, h, abT, ab, bias2)
    return out


def _mm_bias_kernel(x_ref, w_ref, b_ref, o_ref):
    o_ref[...] = _dot(x_ref[...], w_ref[...]) + b_ref[...]


def _attn_kernel(q_ref, k_ref, v_ref, o_ref):
    q = q_ref[...]
    k = k_ref[...]
    v = v_ref[...]
    dh = HID // NHEAD
    outs = []
    for hd in range(NHEAD):
        sl = slice(hd * dh, (hd + 1) * dh)
        s = jax.lax.dot_general(q[:, sl], k[:, sl], (((1,), (1,)), ((), ())),
                                preferred_element_type=F32) * 0.125
        e = jnp.exp(s)
        ssum = jnp.sum(e, axis=1, keepdims=True)
        outs.append(_dot(e.astype(jnp.bfloat16),
                         v[:, sl].astype(jnp.bfloat16)) / ssum)
    o_ref[...] = jnp.concatenate(outs, axis=1)


def _ln(x, g, b):
    m = jnp.mean(x, axis=-1, keepdims=True)
    v = jnp.mean((x - m) * (x - m), axis=-1, keepdims=True)
    return (x - m) / jnp.sqrt(v + 1e-5) * g + b


def _post_kernel(x_ref, o_ref, ow_ref, ob_ref, g1_ref, b1_ref, w1_ref,
                 bb1_ref, w2_ref, bb2_ref, g2_ref, b2_ref, win_ref, skw_ref,
                 skb_ref, out_ref, *, skip):
    x = x_ref[...]
    a = _dot(o_ref[...], ow_ref[...]) + ob_ref[...]
    x1 = _ln(x + a, g1_ref[...], b1_ref[...])
    f = jnp.maximum(_dot(x1, w1_ref[...]) + bb1_ref[...], 0.0)
    f = _dot(f, w2_ref[...]) + bb2_ref[...]
    x2 = _ln(x1 + f, g2_ref[...], b2_ref[...])
    if skip:
        x2 = x2 + _dot(win_ref[...], skw_ref[...]) + skb_ref[...]
    out_ref[...] = x2


def _trans_layer(x, p, window, skw, skb, skip):
    qkv = pl.pallas_call(
        _mm_bias_kernel,
        grid=(N // RBI,),
        in_specs=[
            pl.BlockSpec((RBI, HID), lambda r: (r, 0)),
            pl.BlockSpec((HID, 3 * HID), lambda r: (0, 0)),
            pl.BlockSpec((1, 3 * HID), lambda r: (0, 0)),
        ],
        out_specs=pl.BlockSpec((RBI, 3 * HID), lambda r: (r, 0)),
        out_shape=jax.ShapeDtypeStruct((N, 3 * HID), F32),
    )(x, p['in_w'].T, p['in_b'][None, :])
    o = pl.pallas_call(
        _attn_kernel,
        grid=(N // RBI,),
        in_specs=[
            pl.BlockSpec((RBI, HID), lambda r: (r, 0)),
            pl.BlockSpec((N, HID), lambda r: (0, 1)),
            pl.BlockSpec((N, HID), lambda r: (0, 2)),
        ],
        out_specs=pl.BlockSpec((RBI, HID), lambda r: (r, 0)),
        out_shape=jax.ShapeDtypeStruct((N, HID), F32),
    )(qkv, qkv, qkv)
    out = pl.pallas_call(
        functools.partial(_post_kernel, skip=skip),
        grid=(N // RBI,),
        in_specs=[
            pl.BlockSpec((RBI, HID), lambda r: (r, 0)),
            pl.BlockSpec((RBI, HID), lambda r: (r, 0)),
            pl.BlockSpec((HID, HID), lambda r: (0, 0)),
            pl.BlockSpec((1, HID), lambda r: (0, 0)),
            pl.BlockSpec((1, HID), lambda r: (0, 0)),
            pl.BlockSpec((1, HID), lambda r: (0, 0)),
            pl.BlockSpec((HID, DFF), lambda r: (0, 0)),
            pl.BlockSpec((1, DFF), lambda r: (0, 0)),
            pl.BlockSpec((DFF, HID), lambda r: (0, 0)),
            pl.BlockSpec((1, HID), lambda r: (0, 0)),
            pl.BlockSpec((1, HID), lambda r: (0, 0)),
            pl.BlockSpec((1, HID), lambda r: (0, 0)),
            pl.BlockSpec((RBI, IN_C), lambda r: (r, 0)),
            pl.BlockSpec((IN_C, HID), lambda r: (0, 0)),
            pl.BlockSpec((1, HID), lambda r: (0, 0)),
        ],
        out_specs=pl.BlockSpec((RBI, HID), lambda r: (r, 0)),
        out_shape=jax.ShapeDtypeStruct((N, HID), F32),
    )(x, o, p['out_w'].T, p['out_b'][None, :], p['ln1_g'][None, :],
      p['ln1_b'][None, :], p['l1_w'].T, p['l1_b'][None, :], p['l2_w'].T,
      p['l2_b'][None, :], p['ln2_g'][None, :], p['ln2_b'][None, :],
      window, skw, skb)
    return out


def kernel(window, params):
    x = window
    gl_w = jnp.stack(params['gl_W'])
    nl = gl_w.shape[0]
    emb = pl.pallas_call(
        _emb_kernel,
        grid=(nl,),
        in_specs=[
            pl.BlockSpec((N, IN_C), lambda l: (0, 0)),
            pl.BlockSpec((1, IN_C, HID), lambda l: (l, 0, 0)),
        ],
        out_specs=pl.BlockSpec((1, N, HID), lambda l: (l, 0, 0)),
        out_shape=jax.ShapeDtypeStruct((nl, N, HID), F32),
    )(x, gl_w)
    msk = pl.pallas_call(
        _thresh_kernel,
        grid=(nl, N // RB),
        in_specs=[
            pl.BlockSpec((1, RB, HID), lambda l, r: (l, r, 0)),
            pl.BlockSpec((1, N, HID), lambda l, r: (l, 0, 0)),
        ],
        out_specs=pl.BlockSpec((1, RB, N), lambda l, r: (l, r, 0)),
        out_shape=jax.ShapeDtypeStruct((nl, N, N), jnp.int8),
    )(emb, emb)

    h = x
    for i, p in enumerate(params['enc']):
        h = _gat_layer(h, msk, i, p, HEADS, HID // HEADS, relu=True)

    skw = params['skip_w'].T
    skb = params['skip_b'][None, :]
    ht = h
    for li, p in enumerate(params['trans']):
        ht = _trans_layer(ht, p, window, skw, skb,
                          skip=(li == len(params['trans']) - 1))

    d = ht
    dec = params['dec']
    for i in range(len(dec) - 1):
        d = _gat_layer(d, msk, 3 + i, dec[i], HEADS, HID // HEADS, relu=True)
    return _gat_layer(d, msk, 3 + len(dec) - 1, dec[-1], 1, OUT_C,
                      relu=False, t_out=True)


# speedup vs baseline: 72.4694x; 1.0276x over previous
import functools

import jax
import jax.numpy as jnp
from jax.experimental import pallas as pl

N = 2048
IN_C = 128
HID = 256
OUT_C = 128
HEADS = 4
TOP_K = 16
NHEAD = 4
DFF = 512

RB = 1024
RBI = 512

F32 = jnp.float32


def _dot(a, b):
    return jnp.dot(a, b, preferred_element_type=F32)


def _emb_kernel(x_ref, w_ref, out_ref):
    out_ref[0] = jnp.tanh(_dot(x_ref[...], w_ref[0]))


def _thresh_kernel(embr_ref, embf_ref, msk_ref):
    er = embr_ref[0]
    ef = embf_ref[0]
    sim = jax.lax.dot_general(er, ef, (((1,), (1,)), ((), ())),
                              preferred_element_type=F32)
    a = sim[:, 0:128]
    b = jnp.full((RB, 128), -jnp.inf, F32)
    c = b
    for ci in range(1, N // 128):
        x = sim[:, ci * 128:(ci + 1) * 128]
        lo = jnp.minimum(a, x)
        a = jnp.maximum(a, x)
        lo2 = jnp.minimum(b, lo)
        b = jnp.maximum(b, lo)
        c = jnp.maximum(c, lo2)
    w = jnp.concatenate([a, b, c], axis=1)
    m = None
    m_prev = None
    for k in range(TOP_K + 1):
        m_prev = m
        m = jnp.max(w, axis=1, keepdims=True)
        if k < TOP_K:
            w = jnp.where(w < m, w, -jnp.inf)
    t = (m_prev + m) * 0.5
    msk_ref[0] = (sim >= t).astype(jnp.int8)


def _gat_in_kernel(x_ref, w_ref, a_ref, h_ref, ab_ref, abT_ref):
    h = _dot(x_ref[...], w_ref[...])
    h_ref[...] = h
    ab = _dot(h, a_ref[...])
    ab_ref[...] = ab
    abT_ref[...] = ab.T


def _gat_agg_kernel(msk_ref, h_ref, abT_ref, ab_ref, b_ref, o_ref, *,
                    heads, ch, relu, t_out=False):
    i = pl.program_id(0)
    cols = jax.lax.broadcasted_iota(jnp.int32, (RB, N), 1)
    rowid = i * RB + jax.lax.broadcasted_iota(jnp.int32, (RB, 1), 0)
    B = msk_ref[0].astype(F32) + (cols == rowid).astype(F32)
    hf = h_ref[...]
    outs = []
    for hd in range(heads):
        asrc = abT_ref[hd:hd + 1, :]
        adst = ab_ref[:, 64 + hd:65 + hd]
        ea, ea5 = jnp.exp(asrc), jnp.exp(asrc * 0.2)
        eb, eb5 = jnp.exp(adst), jnp.exp(adst * 0.2)
        e = jnp.where(asrc >= -adst, ea * eb, ea5 * eb5) * B
        ssum = jnp.sum(e, axis=1, keepdims=True)
        outs.append(_dot(e, hf[:, hd * ch:(hd + 1) * ch]) / (ssum + 1e-16))
    o = jnp.concatenate(outs, axis=1) if heads > 1 else outs[0]
    o = o + b_ref[...]
    if relu:
        o = jnp.maximum(o, 0.0)
    if t_out:
        o_ref[...] = o.T
    else:
        o_ref[...] = o


def _gat_layer(x, msk, li, p, heads, ch, relu, t_out=False):
    cin = x.shape[1]
    cout = heads * ch
    att_src = p['att_src']
    att_dst = p['att_dst']
    amat = jnp.zeros((cout, 128), F32)
    for hd in range(heads):
        amat = amat.at[hd * ch:(hd + 1) * ch, hd].set(att_src[hd])
        amat = amat.at[hd * ch:(hd + 1) * ch, 64 + hd].set(att_dst[hd])
    h, ab, abT = pl.pallas_call(
        _gat_in_kernel,
        grid=(N // RBI,),
        in_specs=[
            pl.BlockSpec((RBI, cin), lambda r: (r, 0)),
            pl.BlockSpec((cin, cout), lambda r: (0, 0)),
            pl.BlockSpec((cout, 128), lambda r: (0, 0)),
        ],
        out_specs=[
            pl.BlockSpec((RBI, cout), lambda r: (r, 0)),
            pl.BlockSpec((RBI, 128), lambda r: (r, 0)),
            pl.BlockSpec((128, RBI), lambda r: (0, r)),
        ],
        out_shape=[
            jax.ShapeDtypeStruct((N, cout), F32),
            jax.ShapeDtypeStruct((N, 128), F32),
            jax.ShapeDtypeStruct((128, N), F32),
        ],
    )(x, p['W'], amat)
    bias2 = p['bias'][None, :]
    if t_out:
        ospec = pl.BlockSpec((cout, RB), lambda r: (0, r))
        oshape = jax.ShapeDtypeStruct((cout, N), F32)
    else:
        ospec = pl.BlockSpec((RB, cout), lambda r: (r, 0))
        oshape = jax.ShapeDtypeStruct((N, cout), F32)
    out = pl.pallas_call(
        functools.partial(_gat_agg_kernel, heads=heads, ch=ch, relu=relu,
                          t_out=t_out),
        grid=(N // RB,),
        in_specs=[
            pl.BlockSpec((1, RB, N), lambda r: (li, r, 0)),
            pl.BlockSpec((N, cout), lambda r: (0, 0)),
            pl.BlockSpec((128, N), lambda r: (0, 0)),
            pl.BlockSpec((RB, 128), lambda r: (r, 0)),
            pl.BlockSpec((1, cout), lambda r: (0, 0)),
        ],
        out_specs=ospec,
        out_shape=oshape,
    )(msk, h, abT, ab, bias2)
    return out


def _mm_bias_kernel(x_ref, w_ref, b_ref, o_ref):
    o_ref[...] = _dot(x_ref[...], w_ref[...]) + b_ref[...]


def _attn_kernel(q_ref, k_ref, v_ref, o_ref):
    q = q_ref[...]
    k = k_ref[...]
    v = v_ref[...]
    dh = HID // NHEAD
    outs = []
    for hd in range(NHEAD):
        sl = slice(hd * dh, (hd + 1) * dh)
        s = jax.lax.dot_general(q[:, sl], k[:, sl], (((1,), (1,)), ((), ())),
                                preferred_element_type=F32) * 0.125
        e = jnp.exp(s)
        ssum = jnp.sum(e, axis=1, keepdims=True)
        outs.append(_dot(e.astype(jnp.bfloat16),
                         v[:, sl].astype(jnp.bfloat16)) / ssum)
    o_ref[...] = jnp.concatenate(outs, axis=1)


def _ln(x, g, b):
    m = jnp.mean(x, axis=-1, keepdims=True)
    v = jnp.mean((x - m) * (x - m), axis=-1, keepdims=True)
    return (x - m) / jnp.sqrt(v + 1e-5) * g + b


def _post_kernel(x_ref, o_ref, ow_ref, ob_ref, g1_ref, b1_ref, w1_ref,
                 bb1_ref, w2_ref, bb2_ref, g2_ref, b2_ref, win_ref, skw_ref,
                 skb_ref, out_ref, *, skip):
    x = x_ref[...]
    a = _dot(o_ref[...], ow_ref[...]) + ob_ref[...]
    x1 = _ln(x + a, g1_ref[...], b1_ref[...])
    f = jnp.maximum(_dot(x1, w1_ref[...]) + bb1_ref[...], 0.0)
    f = _dot(f, w2_ref[...]) + bb2_ref[...]
    x2 = _ln(x1 + f, g2_ref[...], b2_ref[...])
    if skip:
        x2 = x2 + _dot(win_ref[...], skw_ref[...]) + skb_ref[...]
    out_ref[...] = x2


def _trans_layer(x, p, window, skw, skb, skip):
    qkv = pl.pallas_call(
        _mm_bias_kernel,
        grid=(N // RBI,),
        in_specs=[
            pl.BlockSpec((RBI, HID), lambda r: (r, 0)),
            pl.BlockSpec((HID, 3 * HID), lambda r: (0, 0)),
            pl.BlockSpec((1, 3 * HID), lambda r: (0, 0)),
        ],
        out_specs=pl.BlockSpec((RBI, 3 * HID), lambda r: (r, 0)),
        out_shape=jax.ShapeDtypeStruct((N, 3 * HID), F32),
    )(x, p['in_w'].T, p['in_b'][None, :])
    o = pl.pallas_call(
        _attn_kernel,
        grid=(N // RBI,),
        in_specs=[
            pl.BlockSpec((RBI, HID), lambda r: (r, 0)),
            pl.BlockSpec((N, HID), lambda r: (0, 1)),
            pl.BlockSpec((N, HID), lambda r: (0, 2)),
        ],
        out_specs=pl.BlockSpec((RBI, HID), lambda r: (r, 0)),
        out_shape=jax.ShapeDtypeStruct((N, HID), F32),
    )(qkv, qkv, qkv)
    out = pl.pallas_call(
        functools.partial(_post_kernel, skip=skip),
        grid=(N // RBI,),
        in_specs=[
            pl.BlockSpec((RBI, HID), lambda r: (r, 0)),
            pl.BlockSpec((RBI, HID), lambda r: (r, 0)),
            pl.BlockSpec((HID, HID), lambda r: (0, 0)),
            pl.BlockSpec((1, HID), lambda r: (0, 0)),
            pl.BlockSpec((1, HID), lambda r: (0, 0)),
            pl.BlockSpec((1, HID), lambda r: (0, 0)),
            pl.BlockSpec((HID, DFF), lambda r: (0, 0)),
            pl.BlockSpec((1, DFF), lambda r: (0, 0)),
            pl.BlockSpec((DFF, HID), lambda r: (0, 0)),
            pl.BlockSpec((1, HID), lambda r: (0, 0)),
            pl.BlockSpec((1, HID), lambda r: (0, 0)),
            pl.BlockSpec((1, HID), lambda r: (0, 0)),
            pl.BlockSpec((RBI, IN_C), lambda r: (r, 0)),
            pl.BlockSpec((IN_C, HID), lambda r: (0, 0)),
            pl.BlockSpec((1, HID), lambda r: (0, 0)),
        ],
        out_specs=pl.BlockSpec((RBI, HID), lambda r: (r, 0)),
        out_shape=jax.ShapeDtypeStruct((N, HID), F32),
    )(x, o, p['out_w'].T, p['out_b'][None, :], p['ln1_g'][None, :],
      p['ln1_b'][None, :], p['l1_w'].T, p['l1_b'][None, :], p['l2_w'].T,
      p['l2_b'][None, :], p['ln2_g'][None, :], p['ln2_b'][None, :],
      window, skw, skb)
    return out


def kernel(window, params):
    x = window
    gl_w = jnp.stack(params['gl_W'])
    nl = gl_w.shape[0]
    emb = pl.pallas_call(
        _emb_kernel,
        grid=(nl,),
        in_specs=[
            pl.BlockSpec((N, IN_C), lambda l: (0, 0)),
            pl.BlockSpec((1, IN_C, HID), lambda l: (l, 0, 0)),
        ],
        out_specs=pl.BlockSpec((1, N, HID), lambda l: (l, 0, 0)),
        out_shape=jax.ShapeDtypeStruct((nl, N, HID), F32),
    )(x, gl_w)
    msk = pl.pallas_call(
        _thresh_kernel,
        grid=(nl, N // RB),
        in_specs=[
            pl.BlockSpec((1, RB, HID), lambda l, r: (l, r, 0)),
            pl.BlockSpec((1, N, HID), lambda l, r: (l, 0, 0)),
        ],
        out_specs=pl.BlockSpec((1, RB, N), lambda l, r: (l, r, 0)),
        out_shape=jax.ShapeDtypeStruct((nl, N, N), jnp.int8),
    )(emb, emb)

    h = x
    for i, p in enumerate(params['enc']):
        h = _gat_layer(h, msk, i, p, HEADS, HID // HEADS, relu=True)

    skw = params['skip_w'].T
    skb = params['skip_b'][None, :]
    ht = h
    for li, p in enumerate(params['trans']):
        ht = _trans_layer(ht, p, window, skw, skb,
                          skip=(li == len(params['trans']) - 1))

    d = ht
    dec = params['dec']
    for i in range(len(dec) - 1):
        d = _gat_layer(d, msk, 3 + i, dec[i], HEADS, HID // HEADS, relu=True)
    return _gat_layer(d, msk, 3 + len(dec) - 1, dec[-1], 1, OUT_C,
                      relu=False, t_out=True)
